# Initial kernel scaffold; baseline (speedup 1.0000x reference)
import functools

import jax
import jax.numpy as jnp
from jax import lax
from jax.experimental import pallas as pl
from jax.experimental.pallas import tpu as pltpu
from jax.experimental.pallas import tpu_sc as plsc

N = 10000
E = 320000
LANES = 128
ROWS = E // LANES
ROWS_PAD = 2560
NTILES = 16
ROWS_PER_TILE = ROWS_PAD // NTILES
SB = 4
NSB = ROWS_PER_TILE // SB
ACC_ROWS = N + 16
DEG_PAD = 10240


def _sync_copy(src, dst, *, add=False):
    def _inner(sem):
        desc = pltpu.make_async_copy(src, dst, sem)
        desc.start(add=add)
        desc.wait()
    pl.run_scoped(_inner, sem=pltpu.SemaphoreType.DMA(()))


def _sc_mesh():
    return plsc.VectorSubcoreMesh(core_axis_name="c", subcore_axis_name="s")


def _deg_body(ei, out, degs, degd, sbuf, dbuf, ones, zbuf):
    c = lax.axis_index("c")
    s = lax.axis_index("s")
    o16 = jnp.ones((16,), jnp.float32)
    z16 = jnp.zeros((16,), jnp.float32)
    for i in range(LANES // 16):
        ones[pl.ds(i * 16, 16)] = o16

    def _zb(i, carry):
        zbuf[pl.ds(i * 16, 16)] = z16
        return carry
    lax.fori_loop(0, 640 // 16, _zb, 0)
    _sync_copy(zbuf, degs.at[pl.ds(s * 640, 640)])
    _sync_copy(zbuf, degd.at[pl.ds(s * 640, 640)])
    plsc.subcore_barrier()

    def _sb(i, carry):
        r0 = s * ROWS_PER_TILE + i * SB
        _sync_copy(ei.at[c, 0, pl.ds(r0, SB)], sbuf)
        _sync_copy(ei.at[c, 1, pl.ds(r0, SB)], dbuf)
        for j in range(SB):
            _sync_copy(ones, degs.at[sbuf.at[j]], add=True)
            _sync_copy(ones, degd.at[dbuf.at[j]], add=True)
        return carry
    lax.fori_loop(0, NSB, _sb, 0)
    plsc.subcore_barrier()
    _sync_copy(degs.at[pl.ds(s * 640, 640)], out.at[c, 0, pl.ds(s * 640, 640)])
    _sync_copy(degd.at[pl.ds(s * 640, 640)], out.at[c, 1, pl.ds(s * 640, 640)])


@jax.jit
def _deg_kernel(ei_deg):
    return pl.kernel(
        _deg_body,
        out_type=jax.ShapeDtypeStruct((2, 2, DEG_PAD), jnp.float32),
        mesh=_sc_mesh(),
        scratch_types=[
            pltpu.VMEM_SHARED((DEG_PAD,), jnp.float32),
            pltpu.VMEM_SHARED((DEG_PAD,), jnp.float32),
            pltpu.VMEM((SB, LANES), jnp.int32),
            pltpu.VMEM((SB, LANES), jnp.int32),
            pltpu.VMEM((LANES,), jnp.float32),
            pltpu.VMEM((640,), jnp.float32),
        ],
    )(ei_deg)


def _msg_body(w, h, srcx, dst2, out, acc, sbuf, dbuf, rows0):
    c = lax.axis_index("c")
    s = lax.axis_index("s")
    z16 = jnp.zeros((16,), jnp.float32)

    def _zr(i, carry):
        for k in range(w // 16):
            rows0[i, pl.ds(k * 16, 16)] = z16
        return carry
    lax.fori_loop(0, LANES, _zr, 0)
    z0 = s * (ACC_ROWS // NTILES)
    nfull = (ACC_ROWS // NTILES) // LANES
    rem = (ACC_ROWS // NTILES) % LANES
    for i in range(nfull):
        _sync_copy(rows0, acc.at[pl.ds(z0 + i * LANES, LANES)])
    _sync_copy(rows0.at[pl.ds(0, rem)], acc.at[pl.ds(z0 + nfull * LANES, rem)])
    plsc.subcore_barrier()

    def _sb(i, carry):
        r0 = s * ROWS_PER_TILE + i * SB
        _sync_copy(srcx.at[c, pl.ds(r0, SB)], sbuf)
        _sync_copy(dst2.at[pl.ds(r0, SB)], dbuf)
        for j in range(SB):
            _sync_copy(h.at[sbuf.at[j]], rows0)
            _sync_copy(rows0, acc.at[dbuf.at[j]], add=True)
        return carry
    lax.fori_loop(0, NSB, _sb, 0)
    plsc.subcore_barrier()
    wb = N // NTILES
    _sync_copy(acc.at[pl.ds(s * wb, wb)], out.at[pl.ds(c * N + s * wb, wb)])


@functools.partial(jax.jit, static_argnums=0)
def _msg_kernel(w, h, srcx, dst2):
    body = functools.partial(_msg_body, w)
    return pl.kernel(
        body,
        out_type=jax.ShapeDtypeStruct((2 * N, w), jnp.float32),
        mesh=_sc_mesh(),
        scratch_types=[
            pltpu.VMEM_SHARED((ACC_ROWS, w), jnp.float32),
            pltpu.VMEM((SB, LANES), jnp.int32),
            pltpu.VMEM((SB, LANES), jnp.int32),
            pltpu.VMEM((LANES, w), jnp.float32),
        ],
    )(h, srcx, dst2)


def _norm_body(d_ref, o_ref):
    d = d_ref[...]
    o_ref[...] = jnp.where(d > 0, lax.rsqrt(d), 0.0)


@jax.jit
def _norm_kernel(degs):
    return pl.pallas_call(
        _norm_body,
        out_shape=jax.ShapeDtypeStruct((2, 2, DEG_PAD), jnp.float32),
    )(degs)


BN = 2000
NB = N // BN


def _lin1_body(x_ref, n_ref, w_ref, o_ref):
    o_ref[...] = jnp.dot(x_ref[...] * n_ref[...], w_ref[...],
                         preferred_element_type=jnp.float32)


@jax.jit
def _lin1_kernel(x, nout, W1):
    f = W1.shape[0]
    return pl.pallas_call(
        _lin1_body,
        grid=(2, NB),
        in_specs=[
            pl.BlockSpec((BN, f), lambda c, n: (n, 0)),
            pl.BlockSpec((BN, 1), lambda c, n: (n, 0)),
            pl.BlockSpec((f, 128), lambda c, n: (0, c)),
        ],
        out_specs=pl.BlockSpec((BN, 128), lambda c, n: (c * NB + n, 0)),
        out_shape=jax.ShapeDtypeStruct((2 * N, 128), jnp.float32),
    )(x, nout, W1)


def _mid_body(wh, a0_ref, a1_ref, ni_ref, no_ref, b_ref, w_ref, o_ref):
    ni = ni_ref[...]
    no = no_ref[...]
    b = b_ref[...]
    t0 = jnp.maximum(a0_ref[...] * ni + b[:, :128], 0.0) * no
    t1 = jnp.maximum(a1_ref[...] * ni + b[:, 128:], 0.0) * no
    t = jnp.concatenate([t0, t1], axis=1)
    o_ref[...] = jnp.dot(t, w_ref[...], preferred_element_type=jnp.float32)


@functools.partial(jax.jit, static_argnums=0)
def _mid_kernel(wh, a0, a1, nin, nout, b, W):
    body = functools.partial(_mid_body, wh)
    return pl.pallas_call(
        body,
        grid=(2, NB),
        in_specs=[
            pl.BlockSpec((BN, 128), lambda c, n: (n, 0)),
            pl.BlockSpec((BN, 128), lambda c, n: (n, 0)),
            pl.BlockSpec((BN, 1), lambda c, n: (n, 0)),
            pl.BlockSpec((BN, 1), lambda c, n: (n, 0)),
            pl.BlockSpec((1, 256), lambda c, n: (0, 0)),
            pl.BlockSpec((256, wh), lambda c, n: (0, c)),
        ],
        out_specs=pl.BlockSpec((BN, wh), lambda c, n: (c * NB + n, 0)),
        out_shape=jax.ShapeDtypeStruct((2 * N, wh), jnp.float32),
    )(a0, a1, nin, nout, b, W)


def _fin_body(a0_ref, a1_ref, ni_ref, b_ref, o_ref):
    ni = ni_ref[...]
    b = b_ref[...]
    t = jnp.concatenate([a0_ref[...], a1_ref[...]], axis=1)
    o_ref[...] = t * ni + b


@jax.jit
def _fin_kernel(a0, a1, nin, b):
    return pl.pallas_call(
        _fin_body,
        grid=(NB,),
        in_specs=[
            pl.BlockSpec((BN, 32), lambda n: (n, 0)),
            pl.BlockSpec((BN, 32), lambda n: (n, 0)),
            pl.BlockSpec((BN, 1), lambda n: (n, 0)),
            pl.BlockSpec((1, 64), lambda n: (0, 0)),
        ],
        out_specs=pl.BlockSpec((BN, 64), lambda n: (n, 0)),
        out_shape=jax.ShapeDtypeStruct((N, 64), jnp.float32),
    )(a0, a1, nin, b)


NPADE = ROWS_PAD * LANES - E


def _prep_graph(edge_index):
    src = edge_index[0].astype(jnp.int32)
    dst = edge_index[1].astype(jnp.int32)
    i = jnp.arange(NPADE, dtype=jnp.int32)
    src_p = jnp.concatenate([src, (i * 97) % N])
    dst_p = jnp.concatenate([dst, N + (i % 16)])
    srcx = jnp.stack([src_p, src_p + N]).reshape(2, ROWS_PAD, LANES)
    dst2 = dst_p.reshape(ROWS_PAD, LANES)
    src_d = jnp.concatenate([src, N + (i % 240)]).reshape(ROWS_PAD, LANES)
    dst_d = jnp.concatenate([dst, N + (i % 240)]).reshape(ROWS_PAD, LANES)
    return srcx, dst2, jnp.stack([src_d, dst_d])


def kernel(features, edge_index1, edge_index2, W1, b1, W2, b2, W3, b3):
    srcx1, dst1, deg_ei1 = _prep_graph(edge_index1)
    srcx2, dst2, deg_ei2 = _prep_graph(edge_index2)
    degs = _deg_kernel(jnp.stack([deg_ei1, deg_ei2]))
    norms = _norm_kernel(degs)
    nout1 = norms[0, 0, :N, None]
    nin1 = norms[0, 1, :N, None]
    nout2 = norms[1, 0, :N, None]
    nin2 = norms[1, 1, :N, None]

    h1 = _lin1_kernel(features, nout2, W1)
    agg1 = _msg_kernel(128, h1, srcx2, dst2)
    h2 = _mid_kernel(128, agg1[:N], agg1[N:], nin2, nout1,
                     b1[None, :], W2)
    agg2 = _msg_kernel(128, h2, srcx1, dst1)
    h3 = _mid_kernel(32, agg2[:N], agg2[N:], nin1, nout1,
                     b2[None, :], W3)
    agg3 = _msg_kernel(32, h3, srcx1, dst1)
    return _fin_kernel(agg3[:N], agg3[N:], nin1, b3[None, :])


# speedup vs baseline: 5.0537x; 5.0537x over previous
import functools

import jax
import jax.numpy as jnp
from jax import lax
from jax.experimental import pallas as pl
from jax.experimental.pallas import tpu as pltpu
from jax.experimental.pallas import tpu_sc as plsc

N = 10000
E = 320000
LANES = 128
ROWS = E // LANES
ROWS_PAD = 2560
NTILES = 16
ROWS_PER_TILE = ROWS_PAD // NTILES
SB = 8
NSB = ROWS_PER_TILE // SB
ACC_ROWS = 10240
DEG_PAD = 10240


def _sync_copy(src, dst, *, add=False):
    def _inner(sem):
        desc = pltpu.make_async_copy(src, dst, sem)
        desc.start(add=add)
        desc.wait()
    pl.run_scoped(_inner, sem=pltpu.SemaphoreType.DMA(()))


def _sc_mesh():
    return plsc.VectorSubcoreMesh(core_axis_name="c", subcore_axis_name="s")


def _deg_body(ei, out, degs, degd, sbuf, dbuf, ones, zbuf):
    c = lax.axis_index("c")
    s = lax.axis_index("s")
    o16 = jnp.ones((16,), jnp.float32)
    z16 = jnp.zeros((16,), jnp.float32)
    for i in range(LANES // 16):
        ones[pl.ds(i * 16, 16)] = o16

    def _zb(i, carry):
        zbuf[pl.ds(i * 16, 16)] = z16
        return carry
    lax.fori_loop(0, 640 // 16, _zb, 0)
    _sync_copy(zbuf, degs.at[pl.ds(s * 640, 640)])
    _sync_copy(zbuf, degd.at[pl.ds(s * 640, 640)])
    plsc.subcore_barrier()

    def _sb(i, carry):
        r0 = s * ROWS_PER_TILE + i * SB
        _sync_copy(ei.at[c, 0, pl.ds(r0, SB)], sbuf)
        _sync_copy(ei.at[c, 1, pl.ds(r0, SB)], dbuf)
        for j in range(SB):
            _sync_copy(ones, degs.at[sbuf.at[j]], add=True)
            _sync_copy(ones, degd.at[dbuf.at[j]], add=True)
        return carry
    lax.fori_loop(0, NSB, _sb, 0)
    plsc.subcore_barrier()
    _sync_copy(degs.at[pl.ds(s * 640, 640)],
               out.at[pl.ds((c * 2 + 0) * DEG_PAD + s * 640, 640)])
    _sync_copy(degd.at[pl.ds(s * 640, 640)],
               out.at[pl.ds((c * 2 + 1) * DEG_PAD + s * 640, 640)])


@jax.jit
def _deg_kernel(ei_deg):
    return pl.kernel(
        _deg_body,
        out_type=jax.ShapeDtypeStruct((4 * DEG_PAD,), jnp.float32),
        mesh=_sc_mesh(),
        scratch_types=[
            pltpu.VMEM_SHARED((DEG_PAD,), jnp.float32),
            pltpu.VMEM_SHARED((DEG_PAD,), jnp.float32),
            pltpu.VMEM((SB, LANES), jnp.int32),
            pltpu.VMEM((SB, LANES), jnp.int32),
            pltpu.VMEM((LANES,), jnp.float32),
            pltpu.VMEM((640,), jnp.float32),
        ],
    )(ei_deg)


def _msg_body(w, h, srcx, dst2, out, acc, sbuf, dbuf, rows0):
    c = lax.axis_index("c")
    s = lax.axis_index("s")
    z16 = jnp.zeros((16,), jnp.float32)

    def _zr(i, carry):
        for k in range(w // 16):
            rows0[i, pl.ds(k * 16, 16)] = z16
        return carry
    lax.fori_loop(0, LANES, _zr, 0)
    z0 = s * (ACC_ROWS // NTILES)
    for i in range((ACC_ROWS // NTILES) // LANES):
        _sync_copy(rows0, acc.at[pl.ds(z0 + i * LANES, LANES)])
    plsc.subcore_barrier()

    def _sb(i, carry):
        r0 = s * ROWS_PER_TILE + i * SB
        _sync_copy(srcx.at[c, pl.ds(r0, SB)], sbuf)
        _sync_copy(dst2.at[pl.ds(r0, SB)], dbuf)
        for j in range(SB):
            _sync_copy(h.at[sbuf.at[j]], rows0)
            _sync_copy(rows0, acc.at[dbuf.at[j]], add=True)
        return carry
    lax.fori_loop(0, NSB, _sb, 0)
    plsc.subcore_barrier()
    wb = ACC_ROWS // NTILES
    _sync_copy(acc.at[pl.ds(s * wb, wb)],
               out.at[pl.ds(c * ACC_ROWS + s * wb, wb)])


@functools.partial(jax.jit, static_argnums=0)
def _msg_kernel(w, h, srcx, dst2):
    body = functools.partial(_msg_body, w)
    return pl.kernel(
        body,
        out_type=jax.ShapeDtypeStruct((2 * ACC_ROWS, w), jnp.float32),
        mesh=_sc_mesh(),
        scratch_types=[
            pltpu.VMEM_SHARED((ACC_ROWS, w), jnp.float32),
            pltpu.VMEM((SB, LANES), jnp.int32),
            pltpu.VMEM((SB, LANES), jnp.int32),
            pltpu.VMEM((LANES, w), jnp.float32),
        ],
    )(h, srcx, dst2)


def _msg_es_body(h, srcx, dst2, out, acc, sbuf, dbuf, rows0):
    c = lax.axis_index("c")
    s = lax.axis_index("s")
    z16 = jnp.zeros((16,), jnp.float32)

    def _zr(i, carry):
        for k in range(128 // 16):
            rows0[i, pl.ds(k * 16, 16)] = z16
        return carry
    lax.fori_loop(0, LANES, _zr, 0)
    z0 = s * (ACC_ROWS // NTILES)
    for i in range((ACC_ROWS // NTILES) // LANES):
        _sync_copy(rows0, acc.at[pl.ds(z0 + i * LANES, LANES)])
    plsc.subcore_barrier()

    rpt = ROWS_PAD // 2 // NTILES

    def _sb(i, carry):
        r0 = c * (ROWS_PAD // 2) + s * rpt + i * SB
        _sync_copy(srcx.at[0, pl.ds(r0, SB)], sbuf)
        _sync_copy(dst2.at[pl.ds(r0, SB)], dbuf)
        for j in range(SB):
            _sync_copy(h.at[sbuf.at[j]], rows0)
            _sync_copy(rows0, acc.at[dbuf.at[j]], add=True)
        return carry
    lax.fori_loop(0, rpt // SB, _sb, 0)
    plsc.subcore_barrier()
    wb = ACC_ROWS // NTILES
    _sync_copy(acc.at[pl.ds(s * wb, wb)],
               out.at[pl.ds(c * ACC_ROWS + s * wb, wb)])


@jax.jit
def _msg_es_kernel(h, srcx, dst2):
    return pl.kernel(
        _msg_es_body,
        out_type=jax.ShapeDtypeStruct((2 * ACC_ROWS, 128), jnp.float32),
        mesh=_sc_mesh(),
        scratch_types=[
            pltpu.VMEM_SHARED((ACC_ROWS, 128), jnp.float32),
            pltpu.VMEM((SB, LANES), jnp.int32),
            pltpu.VMEM((SB, LANES), jnp.int32),
            pltpu.VMEM((LANES, 128), jnp.float32),
        ],
    )(h, srcx, dst2)


def _norm_body(d_ref, o_ref):
    d = d_ref[...]
    o_ref[...] = jnp.where(d > 0, lax.rsqrt(d), 0.0)


@jax.jit
def _norm_kernel(degs):
    return pl.pallas_call(
        _norm_body,
        out_shape=jax.ShapeDtypeStruct((4 * DEG_PAD,), jnp.float32),
    )(degs)


BN = 2000
NB = N // BN


def _lin1_body(x_ref, n_ref, w_ref, o_ref):
    o_ref[...] = jnp.dot(x_ref[...] * n_ref[...], w_ref[...],
                         preferred_element_type=jnp.float32)


@jax.jit
def _lin1_kernel(x, nout, W1):
    f = W1.shape[0]
    return pl.pallas_call(
        _lin1_body,
        grid=(2, NB),
        in_specs=[
            pl.BlockSpec((BN, f), lambda c, n: (n, 0)),
            pl.BlockSpec((BN, 1), lambda c, n: (n, 0)),
            pl.BlockSpec((f, 128), lambda c, n: (0, c)),
        ],
        out_specs=pl.BlockSpec((BN, 128), lambda c, n: (c * NB + n, 0)),
        out_shape=jax.ShapeDtypeStruct((2 * N, 128), jnp.float32),
    )(x, nout, W1)


def _mid_body(wh, a0_ref, a1_ref, ni_ref, no_ref, b_ref, w_ref, o_ref):
    ni = ni_ref[...]
    no = no_ref[...]
    b = b_ref[...]
    t0 = jnp.maximum(a0_ref[...] * ni + b[:, :128], 0.0) * no
    t1 = jnp.maximum(a1_ref[...] * ni + b[:, 128:], 0.0) * no
    t = jnp.concatenate([t0, t1], axis=1)
    o_ref[...] = jnp.dot(t, w_ref[...], preferred_element_type=jnp.float32)


@functools.partial(jax.jit, static_argnums=0)
def _mid_kernel(wh, a0, a1, nin, nout, b, W):
    body = functools.partial(_mid_body, wh)
    if wh == 128:
        return pl.pallas_call(
            body,
            grid=(2, NB),
            in_specs=[
                pl.BlockSpec((BN, 128), lambda c, n: (n, 0)),
                pl.BlockSpec((BN, 128), lambda c, n: (n, 0)),
                pl.BlockSpec((BN, 1), lambda c, n: (n, 0)),
                pl.BlockSpec((BN, 1), lambda c, n: (n, 0)),
                pl.BlockSpec((1, 256), lambda c, n: (0, 0)),
                pl.BlockSpec((256, wh), lambda c, n: (0, c)),
            ],
            out_specs=pl.BlockSpec((BN, wh), lambda c, n: (c * NB + n, 0)),
            out_shape=jax.ShapeDtypeStruct((2 * N, wh), jnp.float32),
        )(a0, a1, nin, nout, b, W)
    return pl.pallas_call(
        body,
        grid=(NB,),
        in_specs=[
            pl.BlockSpec((BN, 128), lambda n: (n, 0)),
            pl.BlockSpec((BN, 128), lambda n: (n, 0)),
            pl.BlockSpec((BN, 1), lambda n: (n, 0)),
            pl.BlockSpec((BN, 1), lambda n: (n, 0)),
            pl.BlockSpec((1, 256), lambda n: (0, 0)),
            pl.BlockSpec((256, 2 * wh), lambda n: (0, 0)),
        ],
        out_specs=pl.BlockSpec((BN, 2 * wh), lambda n: (n, 0)),
        out_shape=jax.ShapeDtypeStruct((N, 2 * wh), jnp.float32),
    )(a0, a1, nin, nout, b, W)


def _fin_body(a0_ref, a1_ref, ni_ref, b_ref, o_ref):
    ni = ni_ref[...]
    b = b_ref[...]
    t = (a0_ref[...] + a1_ref[...])[:, :64]
    o_ref[...] = t * ni + b


@jax.jit
def _fin_kernel(a0, a1, nin, b):
    return pl.pallas_call(
        _fin_body,
        grid=(NB,),
        in_specs=[
            pl.BlockSpec((BN, 128), lambda n: (n, 0)),
            pl.BlockSpec((BN, 128), lambda n: (n, 0)),
            pl.BlockSpec((BN, 1), lambda n: (n, 0)),
            pl.BlockSpec((1, 64), lambda n: (0, 0)),
        ],
        out_specs=pl.BlockSpec((BN, 64), lambda n: (n, 0)),
        out_shape=jax.ShapeDtypeStruct((N, 64), jnp.float32),
    )(a0, a1, nin, b)


NPADE = ROWS_PAD * LANES - E


def _prep_graph(edge_index):
    src = edge_index[0].astype(jnp.int32)
    dst = edge_index[1].astype(jnp.int32)
    i = jnp.arange(NPADE, dtype=jnp.int32)
    src_p = jnp.concatenate([src, (i * 97) % N])
    dst_p = jnp.concatenate([dst, N + (i % (ACC_ROWS - N))])
    srcx = jnp.stack([src_p, src_p + N]).reshape(2, ROWS_PAD, LANES)
    dst2 = dst_p.reshape(ROWS_PAD, LANES)
    src_d = jnp.concatenate([src, N + (i % 240)]).reshape(ROWS_PAD, LANES)
    dst_d = jnp.concatenate([dst, N + (i % 240)]).reshape(ROWS_PAD, LANES)
    return srcx, dst2, jnp.stack([src_d, dst_d])


def kernel(features, edge_index1, edge_index2, W1, b1, W2, b2, W3, b3):
    srcx1, dst1, deg_ei1 = _prep_graph(edge_index1)
    srcx2, dst2, deg_ei2 = _prep_graph(edge_index2)
    degs = _deg_kernel(jnp.stack([deg_ei1, deg_ei2]))
    norms = _norm_kernel(degs).reshape(2, 2, DEG_PAD)
    nout1 = norms[0, 0, :N, None]
    nin1 = norms[0, 1, :N, None]
    nout2 = norms[1, 0, :N, None]
    nin2 = norms[1, 1, :N, None]

    A = ACC_ROWS
    h1 = _lin1_kernel(features, nout2, W1)
    agg1 = _msg_kernel(128, h1, srcx2, dst2)
    h2 = _mid_kernel(128, agg1[:N], agg1[A:A + N], nin2, nout1,
                     b1[None, :], W2)
    agg2 = _msg_kernel(128, h2, srcx1, dst1)
    h3 = _mid_kernel(32, agg2[:N], agg2[A:A + N], nin1, nout1,
                     b2[None, :], W3)
    h3p = jnp.pad(h3, ((0, 0), (0, 64)))
    agg3 = _msg_es_kernel(h3p, srcx1, dst1)
    return _fin_kernel(agg3[:N], agg3[A:A + N], nin1, b3[None, :])


# speedup vs baseline: 6.2943x; 1.2455x over previous
import functools

import jax
import jax.numpy as jnp
from jax import lax
from jax.experimental import pallas as pl
from jax.experimental.pallas import tpu as pltpu
from jax.experimental.pallas import tpu_sc as plsc

N = 10000
E = 320000
LANES = 128
ROWS = E // LANES
ROWS_PAD = 2560
NTILES = 16
ROWS_PER_TILE = ROWS_PAD // NTILES
SB = 8
NSB = ROWS_PER_TILE // SB
ACC_ROWS = 10240
DEG_PAD = 10240


def _sync_copy(src, dst, *, add=False):
    def _inner(sem):
        desc = pltpu.make_async_copy(src, dst, sem)
        desc.start(add=add)
        desc.wait()
    pl.run_scoped(_inner, sem=pltpu.SemaphoreType.DMA(()))


def _sc_mesh():
    return plsc.VectorSubcoreMesh(core_axis_name="c", subcore_axis_name="s")


DEG_CH = 8


def _deg_body(ei, out, degs, degd, sidx, didx, ones, zbuf, sem):
    c = lax.axis_index("c")
    s = lax.axis_index("s")
    o16 = jnp.ones((16,), jnp.float32)
    z16 = jnp.zeros((16,), jnp.float32)
    for i in range(LANES // 16):
        ones[pl.ds(i * 16, 16)] = o16

    def _zb(i, carry):
        zbuf[pl.ds(i * 16, 16)] = z16
        return carry
    lax.fori_loop(0, 640 // 16, _zb, 0)
    _sync_copy(zbuf, degs.at[pl.ds(s * 640, 640)])
    _sync_copy(zbuf, degd.at[pl.ds(s * 640, 640)])
    _sync_copy(ei.at[c, 0, pl.ds(s * ROWS_PER_TILE, ROWS_PER_TILE)], sidx)
    _sync_copy(ei.at[c, 1, pl.ds(s * ROWS_PER_TILE, ROWS_PER_TILE)], didx)
    plsc.subcore_barrier()

    def _ch(i, carry):
        j0 = i * DEG_CH
        for j in range(DEG_CH):
            pltpu.make_async_copy(ones, degs.at[sidx.at[j0 + j]], sem
                                  ).start(add=True)
            pltpu.make_async_copy(ones, degd.at[didx.at[j0 + j]], sem
                                  ).start(add=True)
        for j in range(DEG_CH):
            pltpu.make_async_copy(ones, degs.at[sidx.at[j0 + j]], sem).wait()
            pltpu.make_async_copy(ones, degd.at[didx.at[j0 + j]], sem).wait()
        return carry
    lax.fori_loop(0, ROWS_PER_TILE // DEG_CH, _ch, 0)
    plsc.subcore_barrier()
    _sync_copy(degs.at[pl.ds(s * 640, 640)],
               out.at[pl.ds((c * 2 + 0) * DEG_PAD + s * 640, 640)])
    _sync_copy(degd.at[pl.ds(s * 640, 640)],
               out.at[pl.ds((c * 2 + 1) * DEG_PAD + s * 640, 640)])


@jax.jit
def _deg_kernel(ei_deg):
    return pl.kernel(
        _deg_body,
        out_type=jax.ShapeDtypeStruct((4 * DEG_PAD,), jnp.float32),
        mesh=_sc_mesh(),
        scratch_types=[
            pltpu.VMEM_SHARED((DEG_PAD,), jnp.float32),
            pltpu.VMEM_SHARED((DEG_PAD,), jnp.float32),
            pltpu.VMEM((ROWS_PER_TILE, LANES), jnp.int32),
            pltpu.VMEM((ROWS_PER_TILE, LANES), jnp.int32),
            pltpu.VMEM((LANES,), jnp.float32),
            pltpu.VMEM((640,), jnp.float32),
            pltpu.SemaphoreType.DMA(()),
        ],
    )(ei_deg)


EPB = 64
BPG = 2
PAIR = 2 * BPG
BLK = ROWS_PAD * LANES // EPB
BLK_PER_TILE = BLK // NTILES
PH_W = 64
BLK_ES = BLK // 2 // NTILES
PH_ES = 40


def _msg_phase(h, acc, sidx, didx, A, B, sems, nrows):
    gsa, gsb, ssa, ssb = sems

    def _gst(grp, sem, r):
        for b in range(BPG):
            pltpu.make_async_copy(h.at[sidx.at[r + b]], grp[b], sem).start()

    def _gwt(grp, sem):
        for b in range(BPG):
            pltpu.make_async_copy(h.at[sidx.at[0]], grp[b], sem).wait()

    def _sst(grp, sem, r):
        for b in range(BPG):
            pltpu.make_async_copy(grp[b], acc.at[didx.at[r + b]], sem
                                  ).start(add=True)

    def _swt(grp, sem):
        for b in range(BPG):
            pltpu.make_async_copy(grp[b], acc.at[didx.at[0]], sem).wait()

    _gst(A, gsa, 0)
    _gst(B, gsb, BPG)

    def _it(t, carry):
        r = t * PAIR
        _gwt(A, gsa)
        _sst(A, ssa, r)
        _gwt(B, gsb)
        _sst(B, ssb, r + BPG)
        _swt(A, ssa)
        _gst(A, gsa, r + PAIR)
        _swt(B, ssb)
        _gst(B, gsb, r + PAIR + BPG)
        return carry
    lax.fori_loop(0, nrows // PAIR - 1, _it, 0)
    r = nrows - PAIR
    _gwt(A, gsa)
    _sst(A, ssa, r)
    _gwt(B, gsb)
    _sst(B, ssb, r + BPG)
    _swt(A, ssa)
    _swt(B, ssb)


def _zero_acc(acc, zb, w, s):
    z16 = jnp.zeros((16,), jnp.float32)

    def _zr(i, carry):
        for k in range(w // 16):
            zb[i, pl.ds(k * 16, 16)] = z16
        return carry
    lax.fori_loop(0, EPB, _zr, 0)
    z0 = s * (ACC_ROWS // NTILES)
    for i in range((ACC_ROWS // NTILES) // EPB):
        _sync_copy(zb, acc.at[pl.ds(z0 + i * EPB, EPB)])


def _msg_body(w, h, srcx, dst2, out, acc, sidx, didx,
              a0, a1, b0, b1, gsa, gsb, ssa, ssb):
    c = lax.axis_index("c")
    s = lax.axis_index("s")
    _zero_acc(acc, a0, w, s)
    plsc.subcore_barrier()

    def _ph(p, carry):
        r0 = s * BLK_PER_TILE + p * PH_W
        _sync_copy(srcx.at[c, pl.ds(r0, PH_W)], sidx)
        _sync_copy(dst2.at[pl.ds(r0, PH_W)], didx)
        _msg_phase(h, acc, sidx, didx, [a0, a1], [b0, b1],
                   (gsa, gsb, ssa, ssb), PH_W)
        return carry
    lax.fori_loop(0, BLK_PER_TILE // PH_W, _ph, 0)
    plsc.subcore_barrier()
    wb = ACC_ROWS // NTILES
    _sync_copy(acc.at[pl.ds(s * wb, wb)],
               out.at[pl.ds(c * ACC_ROWS + s * wb, wb)])


@functools.partial(jax.jit, static_argnums=0)
def _msg_kernel(w, h, srcx, dst2):
    body = functools.partial(_msg_body, w)
    return pl.kernel(
        body,
        out_type=jax.ShapeDtypeStruct((2 * ACC_ROWS, w), jnp.float32),
        mesh=_sc_mesh(),
        scratch_types=[
            pltpu.VMEM_SHARED((ACC_ROWS, w), jnp.float32),
            pltpu.VMEM((PH_W, EPB), jnp.int32),
            pltpu.VMEM((PH_W, EPB), jnp.int32),
            pltpu.VMEM((EPB, w), jnp.float32),
            pltpu.VMEM((EPB, w), jnp.float32),
            pltpu.VMEM((EPB, w), jnp.float32),
            pltpu.VMEM((EPB, w), jnp.float32),
            pltpu.SemaphoreType.DMA(()),
            pltpu.SemaphoreType.DMA(()),
            pltpu.SemaphoreType.DMA(()),
            pltpu.SemaphoreType.DMA(()),
        ],
    )(h, srcx, dst2)


def _msg_es_body(h, srcx, dst2, out, acc, sidx, didx,
                 a0, a1, b0, b1, gsa, gsb, ssa, ssb):
    c = lax.axis_index("c")
    s = lax.axis_index("s")
    _zero_acc(acc, a0, 128, s)
    plsc.subcore_barrier()
    base = c * (BLK // 2) + s * BLK_ES

    def _ph(p, carry):
        r0 = base + p * PH_ES
        _sync_copy(srcx.at[0, pl.ds(r0, PH_ES)], sidx)
        _sync_copy(dst2.at[pl.ds(r0, PH_ES)], didx)
        _msg_phase(h, acc, sidx, didx, [a0, a1], [b0, b1],
                   (gsa, gsb, ssa, ssb), PH_ES)
        return carry
    lax.fori_loop(0, BLK_ES // PH_ES, _ph, 0)
    plsc.subcore_barrier()
    wb = ACC_ROWS // NTILES
    _sync_copy(acc.at[pl.ds(s * wb, wb)],
               out.at[pl.ds(c * ACC_ROWS + s * wb, wb)])


@jax.jit
def _msg_es_kernel(h, srcx, dst2):
    return pl.kernel(
        _msg_es_body,
        out_type=jax.ShapeDtypeStruct((2 * ACC_ROWS, 128), jnp.float32),
        mesh=_sc_mesh(),
        scratch_types=[
            pltpu.VMEM_SHARED((ACC_ROWS, 128), jnp.float32),
            pltpu.VMEM((PH_ES, EPB), jnp.int32),
            pltpu.VMEM((PH_ES, EPB), jnp.int32),
            pltpu.VMEM((EPB, 128), jnp.float32),
            pltpu.VMEM((EPB, 128), jnp.float32),
            pltpu.VMEM((EPB, 128), jnp.float32),
            pltpu.VMEM((EPB, 128), jnp.float32),
            pltpu.SemaphoreType.DMA(()),
            pltpu.SemaphoreType.DMA(()),
            pltpu.SemaphoreType.DMA(()),
            pltpu.SemaphoreType.DMA(()),
        ],
    )(h, srcx, dst2)


def _norm_body(d_ref, o_ref):
    d = d_ref[...]
    o_ref[...] = jnp.where(d > 0, lax.rsqrt(d), 0.0)


@jax.jit
def _norm_kernel(degs):
    return pl.pallas_call(
        _norm_body,
        out_shape=jax.ShapeDtypeStruct((4 * DEG_PAD,), jnp.float32),
    )(degs)


BN = 2000
NB = N // BN


def _lin1_body(x_ref, n_ref, w_ref, o_ref):
    o_ref[...] = jnp.dot(x_ref[...] * n_ref[...], w_ref[...],
                         preferred_element_type=jnp.float32)


@jax.jit
def _lin1_kernel(x, nout, W1):
    f = W1.shape[0]
    return pl.pallas_call(
        _lin1_body,
        grid=(2, NB),
        in_specs=[
            pl.BlockSpec((BN, f), lambda c, n: (n, 0)),
            pl.BlockSpec((BN, 1), lambda c, n: (n, 0)),
            pl.BlockSpec((f, 128), lambda c, n: (0, c)),
        ],
        out_specs=pl.BlockSpec((BN, 128), lambda c, n: (c * NB + n, 0)),
        out_shape=jax.ShapeDtypeStruct((2 * N, 128), jnp.float32),
    )(x, nout, W1)


def _mid_body(wh, a0_ref, a1_ref, ni_ref, no_ref, b_ref, w_ref, o_ref):
    ni = ni_ref[...]
    no = no_ref[...]
    b = b_ref[...]
    t0 = jnp.maximum(a0_ref[...] * ni + b[:, :128], 0.0) * no
    t1 = jnp.maximum(a1_ref[...] * ni + b[:, 128:], 0.0) * no
    t = jnp.concatenate([t0, t1], axis=1)
    o_ref[...] = jnp.dot(t, w_ref[...], preferred_element_type=jnp.float32)


@functools.partial(jax.jit, static_argnums=0)
def _mid_kernel(wh, a0, a1, nin, nout, b, W):
    body = functools.partial(_mid_body, wh)
    if wh == 128:
        return pl.pallas_call(
            body,
            grid=(2, NB),
            in_specs=[
                pl.BlockSpec((BN, 128), lambda c, n: (n, 0)),
                pl.BlockSpec((BN, 128), lambda c, n: (n, 0)),
                pl.BlockSpec((BN, 1), lambda c, n: (n, 0)),
                pl.BlockSpec((BN, 1), lambda c, n: (n, 0)),
                pl.BlockSpec((1, 256), lambda c, n: (0, 0)),
                pl.BlockSpec((256, wh), lambda c, n: (0, c)),
            ],
            out_specs=pl.BlockSpec((BN, wh), lambda c, n: (c * NB + n, 0)),
            out_shape=jax.ShapeDtypeStruct((2 * N, wh), jnp.float32),
        )(a0, a1, nin, nout, b, W)
    return pl.pallas_call(
        body,
        grid=(NB,),
        in_specs=[
            pl.BlockSpec((BN, 128), lambda n: (n, 0)),
            pl.BlockSpec((BN, 128), lambda n: (n, 0)),
            pl.BlockSpec((BN, 1), lambda n: (n, 0)),
            pl.BlockSpec((BN, 1), lambda n: (n, 0)),
            pl.BlockSpec((1, 256), lambda n: (0, 0)),
            pl.BlockSpec((256, 2 * wh), lambda n: (0, 0)),
        ],
        out_specs=pl.BlockSpec((BN, 2 * wh), lambda n: (n, 0)),
        out_shape=jax.ShapeDtypeStruct((N, 2 * wh), jnp.float32),
    )(a0, a1, nin, nout, b, W)


def _fin_body(a0_ref, a1_ref, ni_ref, b_ref, o_ref):
    ni = ni_ref[...]
    b = b_ref[...]
    t = (a0_ref[...] + a1_ref[...])[:, :64]
    o_ref[...] = t * ni + b


@jax.jit
def _fin_kernel(a0, a1, nin, b):
    return pl.pallas_call(
        _fin_body,
        grid=(NB,),
        in_specs=[
            pl.BlockSpec((BN, 128), lambda n: (n, 0)),
            pl.BlockSpec((BN, 128), lambda n: (n, 0)),
            pl.BlockSpec((BN, 1), lambda n: (n, 0)),
            pl.BlockSpec((1, 64), lambda n: (0, 0)),
        ],
        out_specs=pl.BlockSpec((BN, 64), lambda n: (n, 0)),
        out_shape=jax.ShapeDtypeStruct((N, 64), jnp.float32),
    )(a0, a1, nin, b)


NPADE = ROWS_PAD * LANES - E


def _prep_graph(edge_index):
    src = edge_index[0].astype(jnp.int32)
    dst = edge_index[1].astype(jnp.int32)
    i = jnp.arange(NPADE, dtype=jnp.int32)
    src_p = jnp.concatenate([src, (i * 97) % N])
    dst_p = jnp.concatenate([dst, N + (i % (ACC_ROWS - N))])
    srcx = jnp.stack([src_p, src_p + N]).reshape(2, BLK, EPB)
    dst2 = dst_p.reshape(BLK, EPB)
    src_d = jnp.concatenate([src, N + (i % 240)]).reshape(ROWS_PAD, LANES)
    dst_d = jnp.concatenate([dst, N + (i % 240)]).reshape(ROWS_PAD, LANES)
    return srcx, dst2, jnp.stack([src_d, dst_d])


def kernel(features, edge_index1, edge_index2, W1, b1, W2, b2, W3, b3):
    srcx1, dst1, deg_ei1 = _prep_graph(edge_index1)
    srcx2, dst2, deg_ei2 = _prep_graph(edge_index2)
    degs = _deg_kernel(jnp.stack([deg_ei1, deg_ei2]))
    norms = _norm_kernel(degs).reshape(2, 2, DEG_PAD)
    nout1 = norms[0, 0, :N, None]
    nin1 = norms[0, 1, :N, None]
    nout2 = norms[1, 0, :N, None]
    nin2 = norms[1, 1, :N, None]

    A = ACC_ROWS
    h1 = _lin1_kernel(features, nout2, W1)
    agg1 = _msg_kernel(128, h1, srcx2, dst2)
    h2 = _mid_kernel(128, agg1[:N], agg1[A:A + N], nin2, nout1,
                     b1[None, :], W2)
    agg2 = _msg_kernel(128, h2, srcx1, dst1)
    h3 = _mid_kernel(32, agg2[:N], agg2[A:A + N], nin1, nout1,
                     b2[None, :], W3)
    h3p = jnp.pad(h3, ((0, 0), (0, 64)))
    agg3 = _msg_es_kernel(h3p, srcx1, dst1)
    return _fin_kernel(agg3[:N], agg3[A:A + N], nin1, b3[None, :])


# speedup vs baseline: 7.3375x; 1.1657x over previous
import functools

import jax
import jax.numpy as jnp
from jax import lax
from jax.experimental import pallas as pl
from jax.experimental.pallas import tpu as pltpu
from jax.experimental.pallas import tpu_sc as plsc

N = 10000
E = 320000
LANES = 128
ROWS = E // LANES
ROWS_PAD = 2560
NTILES = 16
ROWS_PER_TILE = ROWS_PAD // NTILES
SB = 8
NSB = ROWS_PER_TILE // SB
ACC_ROWS = 10240
DEG_PAD = 10240


def _sync_copy(src, dst, *, add=False):
    def _inner(sem):
        desc = pltpu.make_async_copy(src, dst, sem)
        desc.start(add=add)
        desc.wait()
    pl.run_scoped(_inner, sem=pltpu.SemaphoreType.DMA(()))


def _sc_mesh():
    return plsc.VectorSubcoreMesh(core_axis_name="c", subcore_axis_name="s")


DEG_CH = 8


def _deg_body(ei, out, degs, degd, sidx, didx, ones, zbuf, sem):
    c = lax.axis_index("c")
    s = lax.axis_index("s")
    o16 = jnp.ones((16,), jnp.float32)
    z16 = jnp.zeros((16,), jnp.float32)
    for i in range(LANES // 16):
        ones[pl.ds(i * 16, 16)] = o16

    def _zb(i, carry):
        zbuf[pl.ds(i * 16, 16)] = z16
        return carry
    lax.fori_loop(0, 640 // 16, _zb, 0)
    _sync_copy(zbuf, degs.at[pl.ds(s * 640, 640)])
    _sync_copy(zbuf, degd.at[pl.ds(s * 640, 640)])
    _sync_copy(ei.at[c, 0, pl.ds(s * ROWS_PER_TILE, ROWS_PER_TILE)], sidx)
    _sync_copy(ei.at[c, 1, pl.ds(s * ROWS_PER_TILE, ROWS_PER_TILE)], didx)
    plsc.subcore_barrier()

    def _ch(i, carry):
        j0 = i * DEG_CH
        for j in range(DEG_CH):
            pltpu.make_async_copy(ones, degs.at[sidx.at[j0 + j]], sem
                                  ).start(add=True)
            pltpu.make_async_copy(ones, degd.at[didx.at[j0 + j]], sem
                                  ).start(add=True)
        for j in range(DEG_CH):
            pltpu.make_async_copy(ones, degs.at[sidx.at[j0 + j]], sem).wait()
            pltpu.make_async_copy(ones, degd.at[didx.at[j0 + j]], sem).wait()
        return carry
    lax.fori_loop(0, ROWS_PER_TILE // DEG_CH, _ch, 0)
    plsc.subcore_barrier()
    _sync_copy(degs.at[pl.ds(s * 640, 640)],
               out.at[pl.ds((c * 2 + 0) * DEG_PAD + s * 640, 640)])
    _sync_copy(degd.at[pl.ds(s * 640, 640)],
               out.at[pl.ds((c * 2 + 1) * DEG_PAD + s * 640, 640)])


@jax.jit
def _deg_kernel(ei_deg):
    return pl.kernel(
        _deg_body,
        out_type=jax.ShapeDtypeStruct((4 * DEG_PAD,), jnp.float32),
        mesh=_sc_mesh(),
        scratch_types=[
            pltpu.VMEM_SHARED((DEG_PAD,), jnp.float32),
            pltpu.VMEM_SHARED((DEG_PAD,), jnp.float32),
            pltpu.VMEM((ROWS_PER_TILE, LANES), jnp.int32),
            pltpu.VMEM((ROWS_PER_TILE, LANES), jnp.int32),
            pltpu.VMEM((LANES,), jnp.float32),
            pltpu.VMEM((640,), jnp.float32),
            pltpu.SemaphoreType.DMA(()),
        ],
    )(ei_deg)


EPB = 32
NG = 8
BLK = ROWS_PAD * LANES // EPB
BLK_PER_TILE = BLK // NTILES
PH_W = 64
BLK_ES = BLK // 2 // NTILES
PH_ES = 64


def _msg_phase(h, acc, sidx, didx, bufs, gsems, ssems, nrows):
    def _gst(g, r):
        pltpu.make_async_copy(h.at[sidx.at[r]], bufs[g], gsems[g]).start()

    def _gwt(g):
        pltpu.make_async_copy(h.at[sidx.at[0]], bufs[g], gsems[g]).wait()

    def _sst(g, r):
        pltpu.make_async_copy(bufs[g], acc.at[didx.at[r]], ssems[g]
                              ).start(add=True)

    def _swt(g):
        pltpu.make_async_copy(bufs[g], acc.at[didx.at[0]], ssems[g]).wait()

    for g in range(NG):
        _gst(g, g)

    def _it(t, carry):
        r = t * NG
        for g in range(NG):
            _gwt(g)
            _sst(g, r + g)
        for g in range(NG):
            _swt(g)
            _gst(g, r + NG + g)
        return carry
    lax.fori_loop(0, nrows // NG - 1, _it, 0)
    r = nrows - NG
    for g in range(NG):
        _gwt(g)
        _sst(g, r + g)
    for g in range(NG):
        _swt(g)


def _zero_acc(acc, zb, w, s):
    z16 = jnp.zeros((16,), jnp.float32)

    def _zr(i, carry):
        for k in range(w // 16):
            zb[i, pl.ds(k * 16, 16)] = z16
        return carry
    lax.fori_loop(0, EPB, _zr, 0)
    z0 = s * (ACC_ROWS // NTILES)
    for i in range((ACC_ROWS // NTILES) // EPB):
        _sync_copy(zb, acc.at[pl.ds(z0 + i * EPB, EPB)])


def _msg_body(w, h, srcx, dst2, out, acc, sidx, didx,
              b0, b1, b2, b3, b4, b5, b6, b7, gsem, ssem):
    c = lax.axis_index("c")
    s = lax.axis_index("s")
    bufs = [b0, b1, b2, b3, b4, b5, b6, b7]
    gsems = [gsem.at[g] for g in range(NG)]
    ssems = [ssem.at[g] for g in range(NG)]
    _zero_acc(acc, b0, w, s)
    plsc.subcore_barrier()

    def _ph(p, carry):
        r0 = s * BLK_PER_TILE + p * PH_W
        _sync_copy(srcx.at[c, pl.ds(r0, PH_W)], sidx)
        _sync_copy(dst2.at[pl.ds(r0, PH_W)], didx)
        _msg_phase(h, acc, sidx, didx, bufs, gsems, ssems, PH_W)
        return carry
    lax.fori_loop(0, BLK_PER_TILE // PH_W, _ph, 0)
    plsc.subcore_barrier()
    wb = ACC_ROWS // NTILES
    _sync_copy(acc.at[pl.ds(s * wb, wb)],
               out.at[pl.ds(c * ACC_ROWS + s * wb, wb)])


@functools.partial(jax.jit, static_argnums=0)
def _msg_kernel(w, h, srcx, dst2):
    body = functools.partial(_msg_body, w)
    return pl.kernel(
        body,
        out_type=jax.ShapeDtypeStruct((2 * ACC_ROWS, w), jnp.float32),
        mesh=_sc_mesh(),
        scratch_types=[
            pltpu.VMEM_SHARED((ACC_ROWS, w), jnp.float32),
            pltpu.VMEM((PH_W, EPB), jnp.int32),
            pltpu.VMEM((PH_W, EPB), jnp.int32),
        ] + [pltpu.VMEM((EPB, w), jnp.float32)] * NG + [
            pltpu.SemaphoreType.DMA((NG,)),
            pltpu.SemaphoreType.DMA((NG,)),
        ],
    )(h, srcx, dst2)


def _msg_es_body(h, srcx, dst2, out, acc, sidx, didx,
                 b0, b1, b2, b3, b4, b5, b6, b7, gsem, ssem):
    c = lax.axis_index("c")
    s = lax.axis_index("s")
    bufs = [b0, b1, b2, b3, b4, b5, b6, b7]
    gsems = [gsem.at[g] for g in range(NG)]
    ssems = [ssem.at[g] for g in range(NG)]
    _zero_acc(acc, b0, 128, s)
    plsc.subcore_barrier()
    base = c * (BLK // 2) + s * BLK_ES

    def _ph(p, carry):
        r0 = base + p * PH_ES
        _sync_copy(srcx.at[0, pl.ds(r0, PH_ES)], sidx)
        _sync_copy(dst2.at[pl.ds(r0, PH_ES)], didx)
        _msg_phase(h, acc, sidx, didx, bufs, gsems, ssems, PH_ES)
        return carry
    lax.fori_loop(0, BLK_ES // PH_ES, _ph, 0)
    plsc.subcore_barrier()
    wb = ACC_ROWS // NTILES
    _sync_copy(acc.at[pl.ds(s * wb, wb)],
               out.at[pl.ds(c * ACC_ROWS + s * wb, wb)])


@jax.jit
def _msg_es_kernel(h, srcx, dst2):
    return pl.kernel(
        _msg_es_body,
        out_type=jax.ShapeDtypeStruct((2 * ACC_ROWS, 128), jnp.float32),
        mesh=_sc_mesh(),
        scratch_types=[
            pltpu.VMEM_SHARED((ACC_ROWS, 128), jnp.float32),
            pltpu.VMEM((PH_ES, EPB), jnp.int32),
            pltpu.VMEM((PH_ES, EPB), jnp.int32),
        ] + [pltpu.VMEM((EPB, 128), jnp.float32)] * NG + [
            pltpu.SemaphoreType.DMA((NG,)),
            pltpu.SemaphoreType.DMA((NG,)),
        ],
    )(h, srcx, dst2)


def _norm_body(d_ref, o_ref):
    d = d_ref[...]
    o_ref[...] = jnp.where(d > 0, lax.rsqrt(d), 0.0)


@jax.jit
def _norm_kernel(degs):
    return pl.pallas_call(
        _norm_body,
        out_shape=jax.ShapeDtypeStruct((4 * DEG_PAD,), jnp.float32),
    )(degs)


BN = 2000
NB = N // BN


def _lin1_body(x_ref, n_ref, w_ref, o_ref):
    o_ref[...] = jnp.dot(x_ref[...] * n_ref[...], w_ref[...],
                         preferred_element_type=jnp.float32)


@jax.jit
def _lin1_kernel(x, nout, W1):
    f = W1.shape[0]
    return pl.pallas_call(
        _lin1_body,
        grid=(2, NB),
        in_specs=[
            pl.BlockSpec((BN, f), lambda c, n: (n, 0)),
            pl.BlockSpec((BN, 1), lambda c, n: (n, 0)),
            pl.BlockSpec((f, 128), lambda c, n: (0, c)),
        ],
        out_specs=pl.BlockSpec((BN, 128), lambda c, n: (c * NB + n, 0)),
        out_shape=jax.ShapeDtypeStruct((2 * N, 128), jnp.float32),
    )(x, nout, W1)


def _mid_body(wh, a0_ref, a1_ref, ni_ref, no_ref, b_ref, w_ref, o_ref):
    ni = ni_ref[...]
    no = no_ref[...]
    b = b_ref[...]
    t0 = jnp.maximum(a0_ref[...] * ni + b[:, :128], 0.0) * no
    t1 = jnp.maximum(a1_ref[...] * ni + b[:, 128:], 0.0) * no
    t = jnp.concatenate([t0, t1], axis=1)
    o_ref[...] = jnp.dot(t, w_ref[...], preferred_element_type=jnp.float32)


@functools.partial(jax.jit, static_argnums=0)
def _mid_kernel(wh, a0, a1, nin, nout, b, W):
    body = functools.partial(_mid_body, wh)
    if wh == 128:
        return pl.pallas_call(
            body,
            grid=(2, NB),
            in_specs=[
                pl.BlockSpec((BN, 128), lambda c, n: (n, 0)),
                pl.BlockSpec((BN, 128), lambda c, n: (n, 0)),
                pl.BlockSpec((BN, 1), lambda c, n: (n, 0)),
                pl.BlockSpec((BN, 1), lambda c, n: (n, 0)),
                pl.BlockSpec((1, 256), lambda c, n: (0, 0)),
                pl.BlockSpec((256, wh), lambda c, n: (0, c)),
            ],
            out_specs=pl.BlockSpec((BN, wh), lambda c, n: (c * NB + n, 0)),
            out_shape=jax.ShapeDtypeStruct((2 * N, wh), jnp.float32),
        )(a0, a1, nin, nout, b, W)
    return pl.pallas_call(
        body,
        grid=(NB,),
        in_specs=[
            pl.BlockSpec((BN, 128), lambda n: (n, 0)),
            pl.BlockSpec((BN, 128), lambda n: (n, 0)),
            pl.BlockSpec((BN, 1), lambda n: (n, 0)),
            pl.BlockSpec((BN, 1), lambda n: (n, 0)),
            pl.BlockSpec((1, 256), lambda n: (0, 0)),
            pl.BlockSpec((256, 2 * wh), lambda n: (0, 0)),
        ],
        out_specs=pl.BlockSpec((BN, 2 * wh), lambda n: (n, 0)),
        out_shape=jax.ShapeDtypeStruct((N, 2 * wh), jnp.float32),
    )(a0, a1, nin, nout, b, W)


def _fin_body(a0_ref, a1_ref, ni_ref, b_ref, o_ref):
    ni = ni_ref[...]
    b = b_ref[...]
    t = (a0_ref[...] + a1_ref[...])[:, :64]
    o_ref[...] = t * ni + b


@jax.jit
def _fin_kernel(a0, a1, nin, b):
    return pl.pallas_call(
        _fin_body,
        grid=(NB,),
        in_specs=[
            pl.BlockSpec((BN, 128), lambda n: (n, 0)),
            pl.BlockSpec((BN, 128), lambda n: (n, 0)),
            pl.BlockSpec((BN, 1), lambda n: (n, 0)),
            pl.BlockSpec((1, 64), lambda n: (0, 0)),
        ],
        out_specs=pl.BlockSpec((BN, 64), lambda n: (n, 0)),
        out_shape=jax.ShapeDtypeStruct((N, 64), jnp.float32),
    )(a0, a1, nin, b)


NPADE = ROWS_PAD * LANES - E


def _prep_graph(edge_index):
    src = edge_index[0].astype(jnp.int32)
    dst = edge_index[1].astype(jnp.int32)
    i = jnp.arange(NPADE, dtype=jnp.int32)
    src_p = jnp.concatenate([src, (i * 97) % N])
    dst_p = jnp.concatenate([dst, N + (i % (ACC_ROWS - N))])
    srcx = jnp.stack([src_p, src_p + N]).reshape(2, BLK, EPB)
    dst2 = dst_p.reshape(BLK, EPB)
    src_d = jnp.concatenate([src, N + (i % 240)]).reshape(ROWS_PAD, LANES)
    dst_d = jnp.concatenate([dst, N + (i % 240)]).reshape(ROWS_PAD, LANES)
    return srcx, dst2, jnp.stack([src_d, dst_d])


def kernel(features, edge_index1, edge_index2, W1, b1, W2, b2, W3, b3):
    srcx1, dst1, deg_ei1 = _prep_graph(edge_index1)
    srcx2, dst2, deg_ei2 = _prep_graph(edge_index2)
    degs = _deg_kernel(jnp.stack([deg_ei1, deg_ei2]))
    norms = _norm_kernel(degs).reshape(2, 2, DEG_PAD)
    nout1 = norms[0, 0, :N, None]
    nin1 = norms[0, 1, :N, None]
    nout2 = norms[1, 0, :N, None]
    nin2 = norms[1, 1, :N, None]

    A = ACC_ROWS
    h1 = _lin1_kernel(features, nout2, W1)
    agg1 = _msg_kernel(128, h1, srcx2, dst2)
    h2 = _mid_kernel(128, agg1[:N], agg1[A:A + N], nin2, nout1,
                     b1[None, :], W2)
    agg2 = _msg_kernel(128, h2, srcx1, dst1)
    h3 = _mid_kernel(32, agg2[:N], agg2[A:A + N], nin1, nout1,
                     b2[None, :], W3)
    h3p = jnp.pad(h3, ((0, 0), (0, 64)))
    agg3 = _msg_es_kernel(h3p, srcx1, dst1)
    return _fin_kernel(agg3[:N], agg3[A:A + N], nin1, b3[None, :])


# speedup vs baseline: 7.3869x; 1.0067x over previous
import functools

import jax
import jax.numpy as jnp
from jax import lax
from jax.experimental import pallas as pl
from jax.experimental.pallas import tpu as pltpu
from jax.experimental.pallas import tpu_sc as plsc

N = 10000
E = 320000
LANES = 128
ROWS = E // LANES
ROWS_PAD = 2560
NTILES = 16
ROWS_PER_TILE = ROWS_PAD // NTILES
SB = 8
NSB = ROWS_PER_TILE // SB
ACC_ROWS = 10240
DEG_PAD = 10240


def _sync_copy(src, dst, *, add=False):
    def _inner(sem):
        desc = pltpu.make_async_copy(src, dst, sem)
        desc.start(add=add)
        desc.wait()
    pl.run_scoped(_inner, sem=pltpu.SemaphoreType.DMA(()))


def _sc_mesh():
    return plsc.VectorSubcoreMesh(core_axis_name="c", subcore_axis_name="s")


DEG_CH = 8


def _deg_body(ei, out, degs, degd, sidx, didx, ones, zbuf, sem):
    c = lax.axis_index("c")
    s = lax.axis_index("s")
    o16 = jnp.ones((16,), jnp.float32)
    z16 = jnp.zeros((16,), jnp.float32)
    for i in range(LANES // 16):
        ones[pl.ds(i * 16, 16)] = o16

    def _zb(i, carry):
        zbuf[pl.ds(i * 16, 16)] = z16
        return carry
    lax.fori_loop(0, 640 // 16, _zb, 0)
    _sync_copy(zbuf, degs.at[pl.ds(s * 640, 640)])
    _sync_copy(zbuf, degd.at[pl.ds(s * 640, 640)])
    _sync_copy(ei.at[c, 0, pl.ds(s * ROWS_PER_TILE, ROWS_PER_TILE)], sidx)
    _sync_copy(ei.at[c, 1, pl.ds(s * ROWS_PER_TILE, ROWS_PER_TILE)], didx)
    plsc.subcore_barrier()

    def _ch(i, carry):
        j0 = i * DEG_CH
        for j in range(DEG_CH):
            pltpu.make_async_copy(ones, degs.at[sidx.at[j0 + j]], sem
                                  ).start(add=True)
            pltpu.make_async_copy(ones, degd.at[didx.at[j0 + j]], sem
                                  ).start(add=True)
        for j in range(DEG_CH):
            pltpu.make_async_copy(ones, degs.at[sidx.at[j0 + j]], sem).wait()
            pltpu.make_async_copy(ones, degd.at[didx.at[j0 + j]], sem).wait()
        return carry
    lax.fori_loop(0, ROWS_PER_TILE // DEG_CH, _ch, 0)
    plsc.subcore_barrier()
    _sync_copy(degs.at[pl.ds(s * 640, 640)],
               out.at[pl.ds((c * 2 + 0) * DEG_PAD + s * 640, 640)])
    _sync_copy(degd.at[pl.ds(s * 640, 640)],
               out.at[pl.ds((c * 2 + 1) * DEG_PAD + s * 640, 640)])


@jax.jit
def _deg_kernel(ei_deg):
    return pl.kernel(
        _deg_body,
        out_type=jax.ShapeDtypeStruct((4 * DEG_PAD,), jnp.float32),
        mesh=_sc_mesh(),
        scratch_types=[
            pltpu.VMEM_SHARED((DEG_PAD,), jnp.float32),
            pltpu.VMEM_SHARED((DEG_PAD,), jnp.float32),
            pltpu.VMEM((ROWS_PER_TILE, LANES), jnp.int32),
            pltpu.VMEM((ROWS_PER_TILE, LANES), jnp.int32),
            pltpu.VMEM((LANES,), jnp.float32),
            pltpu.VMEM((640,), jnp.float32),
            pltpu.SemaphoreType.DMA(()),
        ],
    )(ei_deg)


EPB = 32
NG = 8
BLK = ROWS_PAD * LANES // EPB
BLK_PER_TILE = BLK // NTILES
PH_W = 64
BLK_ES = BLK // 2 // NTILES
PH_ES = 64


def _msg_phase(h, acc, idx, bufs, gsems, ssems, nrows):
    def _gst(g, r):
        pltpu.make_async_copy(h.at[idx.at[r, 0]], bufs[g], gsems[g]).start()

    def _gwt(g):
        pltpu.make_async_copy(h.at[idx.at[0, 0]], bufs[g], gsems[g]).wait()

    def _sst(g, r):
        pltpu.make_async_copy(bufs[g], acc.at[idx.at[r, 1]], ssems[g]
                              ).start(add=True)

    def _swt(g):
        pltpu.make_async_copy(bufs[g], acc.at[idx.at[0, 1]], ssems[g]).wait()

    for g in range(NG):
        _gst(g, g)

    def _it(t, carry):
        r = t * NG
        for g in range(NG):
            _gwt(g)
            _sst(g, r + g)
        for g in range(NG):
            _swt(g)
            _gst(g, r + NG + g)
        return carry
    lax.fori_loop(0, nrows // NG - 1, _it, 0)
    r = nrows - NG
    for g in range(NG):
        _gwt(g)
        _sst(g, r + g)
    for g in range(NG):
        _swt(g)


def _zero_acc(acc, zb, w, s):
    z16 = jnp.zeros((16,), jnp.float32)

    def _zr(i, carry):
        for k in range(w // 16):
            zb[i, pl.ds(k * 16, 16)] = z16
        return carry
    lax.fori_loop(0, EPB, _zr, 0)
    z0 = s * (ACC_ROWS // NTILES)
    for i in range((ACC_ROWS // NTILES) // EPB):
        _sync_copy(zb, acc.at[pl.ds(z0 + i * EPB, EPB)])


def _msg_body(w, h, sdx, out, acc, idx,
              b0, b1, b2, b3, b4, b5, b6, b7, gsem, ssem):
    c = lax.axis_index("c")
    s = lax.axis_index("s")
    bufs = [b0, b1, b2, b3, b4, b5, b6, b7]
    gsems = [gsem.at[g] for g in range(NG)]
    ssems = [ssem.at[g] for g in range(NG)]
    _zero_acc(acc, b0, w, s)
    plsc.subcore_barrier()

    def _ph(p, carry):
        r0 = s * BLK_PER_TILE + p * PH_W
        _sync_copy(sdx.at[c, pl.ds(r0, PH_W)], idx)
        _msg_phase(h, acc, idx, bufs, gsems, ssems, PH_W)
        return carry
    lax.fori_loop(0, BLK_PER_TILE // PH_W, _ph, 0)
    plsc.subcore_barrier()
    wb = ACC_ROWS // NTILES
    _sync_copy(acc.at[pl.ds(s * wb, wb)],
               out.at[pl.ds(c * ACC_ROWS + s * wb, wb)])


@functools.partial(jax.jit, static_argnums=0)
def _msg_kernel(w, h, sdx):
    body = functools.partial(_msg_body, w)
    return pl.kernel(
        body,
        out_type=jax.ShapeDtypeStruct((2 * ACC_ROWS, w), jnp.float32),
        mesh=_sc_mesh(),
        scratch_types=[
            pltpu.VMEM_SHARED((ACC_ROWS, w), jnp.float32),
            pltpu.VMEM((PH_W, 2, EPB), jnp.int32),
        ] + [pltpu.VMEM((EPB, w), jnp.float32)] * NG + [
            pltpu.SemaphoreType.DMA((NG,)),
            pltpu.SemaphoreType.DMA((NG,)),
        ],
    )(h, sdx)


def _msg_es_body(h, sdx, out, acc, idx,
                 b0, b1, b2, b3, b4, b5, b6, b7, gsem, ssem):
    c = lax.axis_index("c")
    s = lax.axis_index("s")
    bufs = [b0, b1, b2, b3, b4, b5, b6, b7]
    gsems = [gsem.at[g] for g in range(NG)]
    ssems = [ssem.at[g] for g in range(NG)]
    _zero_acc(acc, b0, 128, s)
    plsc.subcore_barrier()
    base = c * (BLK // 2) + s * BLK_ES

    def _ph(p, carry):
        r0 = base + p * PH_ES
        _sync_copy(sdx.at[0, pl.ds(r0, PH_ES)], idx)
        _msg_phase(h, acc, idx, bufs, gsems, ssems, PH_ES)
        return carry
    lax.fori_loop(0, BLK_ES // PH_ES, _ph, 0)
    plsc.subcore_barrier()
    wb = ACC_ROWS // NTILES
    _sync_copy(acc.at[pl.ds(s * wb, wb)],
               out.at[pl.ds(c * ACC_ROWS + s * wb, wb)])


@jax.jit
def _msg_es_kernel(h, sdx):
    return pl.kernel(
        _msg_es_body,
        out_type=jax.ShapeDtypeStruct((2 * ACC_ROWS, 128), jnp.float32),
        mesh=_sc_mesh(),
        scratch_types=[
            pltpu.VMEM_SHARED((ACC_ROWS, 128), jnp.float32),
            pltpu.VMEM((PH_ES, 2, EPB), jnp.int32),
        ] + [pltpu.VMEM((EPB, 128), jnp.float32)] * NG + [
            pltpu.SemaphoreType.DMA((NG,)),
            pltpu.SemaphoreType.DMA((NG,)),
        ],
    )(h, sdx)


def _rsqrt0(d):
    return jnp.where(d > 0, lax.rsqrt(d), 0.0)


BN = 2000
NB = N // BN


def _lin1_body(x_ref, n_ref, w_ref, o_ref):
    o_ref[...] = jnp.dot(x_ref[...] * _rsqrt0(n_ref[...]), w_ref[...],
                         preferred_element_type=jnp.float32)


@jax.jit
def _lin1_kernel(x, nout, W1):
    f = W1.shape[0]
    return pl.pallas_call(
        _lin1_body,
        grid=(2, NB),
        in_specs=[
            pl.BlockSpec((BN, f), lambda c, n: (n, 0)),
            pl.BlockSpec((BN, 1), lambda c, n: (n, 0)),
            pl.BlockSpec((f, 128), lambda c, n: (0, c)),
        ],
        out_specs=pl.BlockSpec((BN, 128), lambda c, n: (c * NB + n, 0)),
        out_shape=jax.ShapeDtypeStruct((2 * N, 128), jnp.float32),
    )(x, nout, W1)


def _mid_body(wh, a0_ref, a1_ref, ni_ref, no_ref, b_ref, w_ref, o_ref):
    ni = _rsqrt0(ni_ref[...])
    no = _rsqrt0(no_ref[...])
    b = b_ref[...]
    t0 = jnp.maximum(a0_ref[...] * ni + b[:, :128], 0.0) * no
    t1 = jnp.maximum(a1_ref[...] * ni + b[:, 128:], 0.0) * no
    t = jnp.concatenate([t0, t1], axis=1)
    o_ref[...] = jnp.dot(t, w_ref[...], preferred_element_type=jnp.float32)


@functools.partial(jax.jit, static_argnums=0)
def _mid_kernel(wh, a0, a1, nin, nout, b, W):
    body = functools.partial(_mid_body, wh)
    if wh == 128:
        return pl.pallas_call(
            body,
            grid=(2, NB),
            in_specs=[
                pl.BlockSpec((BN, 128), lambda c, n: (n, 0)),
                pl.BlockSpec((BN, 128), lambda c, n: (n, 0)),
                pl.BlockSpec((BN, 1), lambda c, n: (n, 0)),
                pl.BlockSpec((BN, 1), lambda c, n: (n, 0)),
                pl.BlockSpec((1, 256), lambda c, n: (0, 0)),
                pl.BlockSpec((256, wh), lambda c, n: (0, c)),
            ],
            out_specs=pl.BlockSpec((BN, wh), lambda c, n: (c * NB + n, 0)),
            out_shape=jax.ShapeDtypeStruct((2 * N, wh), jnp.float32),
        )(a0, a1, nin, nout, b, W)
    return pl.pallas_call(
        body,
        grid=(NB,),
        in_specs=[
            pl.BlockSpec((BN, 128), lambda n: (n, 0)),
            pl.BlockSpec((BN, 128), lambda n: (n, 0)),
            pl.BlockSpec((BN, 1), lambda n: (n, 0)),
            pl.BlockSpec((BN, 1), lambda n: (n, 0)),
            pl.BlockSpec((1, 256), lambda n: (0, 0)),
            pl.BlockSpec((256, 2 * wh), lambda n: (0, 0)),
        ],
        out_specs=pl.BlockSpec((BN, 2 * wh), lambda n: (n, 0)),
        out_shape=jax.ShapeDtypeStruct((N, 2 * wh), jnp.float32),
    )(a0, a1, nin, nout, b, W)


def _fin_body(a0_ref, a1_ref, ni_ref, b_ref, o_ref):
    ni = _rsqrt0(ni_ref[...])
    b = b_ref[...]
    t = (a0_ref[...] + a1_ref[...])[:, :64]
    o_ref[...] = t * ni + b


@jax.jit
def _fin_kernel(a0, a1, nin, b):
    return pl.pallas_call(
        _fin_body,
        grid=(NB,),
        in_specs=[
            pl.BlockSpec((BN, 128), lambda n: (n, 0)),
            pl.BlockSpec((BN, 128), lambda n: (n, 0)),
            pl.BlockSpec((BN, 1), lambda n: (n, 0)),
            pl.BlockSpec((1, 64), lambda n: (0, 0)),
        ],
        out_specs=pl.BlockSpec((BN, 64), lambda n: (n, 0)),
        out_shape=jax.ShapeDtypeStruct((N, 64), jnp.float32),
    )(a0, a1, nin, b)


NPADE = ROWS_PAD * LANES - E


def _prep_graph(edge_index):
    src = edge_index[0].astype(jnp.int32)
    dst = edge_index[1].astype(jnp.int32)
    i = jnp.arange(NPADE, dtype=jnp.int32)
    src_p = jnp.concatenate([src, (i * 97) % N])
    dst_p = jnp.concatenate([dst, N + (i % (ACC_ROWS - N))])
    sdx = jnp.stack([
        jnp.stack([src_p.reshape(BLK, EPB), dst_p.reshape(BLK, EPB)], 1),
        jnp.stack([(src_p + N).reshape(BLK, EPB), dst_p.reshape(BLK, EPB)],
                  1)])
    src_d = jnp.concatenate([src, N + (i % 240)]).reshape(ROWS_PAD, LANES)
    dst_d = jnp.concatenate([dst, N + (i % 240)]).reshape(ROWS_PAD, LANES)
    return sdx, jnp.stack([src_d, dst_d])


def kernel(features, edge_index1, edge_index2, W1, b1, W2, b2, W3, b3):
    sdx1, deg_ei1 = _prep_graph(edge_index1)
    sdx2, deg_ei2 = _prep_graph(edge_index2)
    degs = _deg_kernel(jnp.stack([deg_ei1, deg_ei2])).reshape(2, 2, DEG_PAD)
    dout1 = degs[0, 0, :N, None]
    din1 = degs[0, 1, :N, None]
    dout2 = degs[1, 0, :N, None]
    din2 = degs[1, 1, :N, None]

    A = ACC_ROWS
    h1 = _lin1_kernel(features, dout2, W1)
    agg1 = _msg_kernel(128, h1, sdx2)
    h2 = _mid_kernel(128, agg1[:N], agg1[A:A + N], din2, dout1,
                     b1[None, :], W2)
    agg2 = _msg_kernel(128, h2, sdx1)
    h3 = _mid_kernel(32, agg2[:N], agg2[A:A + N], din1, dout1,
                     b2[None, :], W3)
    h3p = jnp.pad(h3, ((0, 0), (0, 64)))
    agg3 = _msg_es_kernel(h3p, sdx1)
    return _fin_kernel(agg3[:N], agg3[A:A + N], din1, b3[None, :])


# speedup vs baseline: 7.5252x; 1.0187x over previous
import functools

import jax
import jax.numpy as jnp
from jax import lax
from jax.experimental import pallas as pl
from jax.experimental.pallas import tpu as pltpu
from jax.experimental.pallas import tpu_sc as plsc

N = 10000
E = 320000
LANES = 128
ROWS = E // LANES
ROWS_PAD = 2560
NTILES = 16
ROWS_PER_TILE = ROWS_PAD // NTILES
SB = 8
NSB = ROWS_PER_TILE // SB
ACC_ROWS = 10240
DEG_PAD = 10240


def _sync_copy(src, dst, *, add=False):
    def _inner(sem):
        desc = pltpu.make_async_copy(src, dst, sem)
        desc.start(add=add)
        desc.wait()
    pl.run_scoped(_inner, sem=pltpu.SemaphoreType.DMA(()))


def _sc_mesh():
    return plsc.VectorSubcoreMesh(core_axis_name="c", subcore_axis_name="s")


DEG_CH = 8


def _deg_body(ei, out, degs, degd, sidx, didx, ones, zbuf, sem):
    c = lax.axis_index("c")
    s = lax.axis_index("s")
    o16 = jnp.ones((16,), jnp.float32)
    z16 = jnp.zeros((16,), jnp.float32)
    for i in range(LANES // 16):
        ones[pl.ds(i * 16, 16)] = o16

    def _zb(i, carry):
        zbuf[pl.ds(i * 16, 16)] = z16
        return carry
    lax.fori_loop(0, 640 // 16, _zb, 0)
    _sync_copy(zbuf, degs.at[pl.ds(s * 640, 640)])
    _sync_copy(zbuf, degd.at[pl.ds(s * 640, 640)])
    _sync_copy(ei.at[c, 0, pl.ds(s * ROWS_PER_TILE, ROWS_PER_TILE)], sidx)
    _sync_copy(ei.at[c, 1, pl.ds(s * ROWS_PER_TILE, ROWS_PER_TILE)], didx)
    plsc.subcore_barrier()

    def _ch(i, carry):
        j0 = i * DEG_CH
        for j in range(DEG_CH):
            pltpu.make_async_copy(ones, degs.at[sidx.at[j0 + j]], sem
                                  ).start(add=True)
            pltpu.make_async_copy(ones, degd.at[didx.at[j0 + j]], sem
                                  ).start(add=True)
        for j in range(DEG_CH):
            pltpu.make_async_copy(ones, degs.at[sidx.at[j0 + j]], sem).wait()
            pltpu.make_async_copy(ones, degd.at[didx.at[j0 + j]], sem).wait()
        return carry
    lax.fori_loop(0, ROWS_PER_TILE // DEG_CH, _ch, 0)
    plsc.subcore_barrier()
    _sync_copy(degs.at[pl.ds(s * 640, 640)],
               out.at[pl.ds((c * 2 + 0) * DEG_PAD + s * 640, 640)])
    _sync_copy(degd.at[pl.ds(s * 640, 640)],
               out.at[pl.ds((c * 2 + 1) * DEG_PAD + s * 640, 640)])


@jax.jit
def _deg_kernel(ei_deg):
    return pl.kernel(
        _deg_body,
        out_type=jax.ShapeDtypeStruct((4 * DEG_PAD,), jnp.float32),
        mesh=_sc_mesh(),
        scratch_types=[
            pltpu.VMEM_SHARED((DEG_PAD,), jnp.float32),
            pltpu.VMEM_SHARED((DEG_PAD,), jnp.float32),
            pltpu.VMEM((ROWS_PER_TILE, LANES), jnp.int32),
            pltpu.VMEM((ROWS_PER_TILE, LANES), jnp.int32),
            pltpu.VMEM((LANES,), jnp.float32),
            pltpu.VMEM((640,), jnp.float32),
            pltpu.SemaphoreType.DMA(()),
        ],
    )(ei_deg)


EPB = 32
NG = 8
BLK = ROWS_PAD * LANES // EPB
BLK_PER_TILE = BLK // NTILES
PH_W = 64
BLK_ES = BLK // 2 // NTILES
PH_ES = 64


def _msg_phase(h, acc, idx, bufs, gsems, ssems, nrows):
    def _gst(g, r):
        pltpu.make_async_copy(h.at[idx.at[r, 0]], bufs[g], gsems[g]).start()

    def _gwt(g):
        pltpu.make_async_copy(h.at[idx.at[0, 0]], bufs[g], gsems[g]).wait()

    def _sst(g, r):
        pltpu.make_async_copy(bufs[g], acc.at[idx.at[r, 1]], ssems[g]
                              ).start(add=True)

    def _swt(g):
        pltpu.make_async_copy(bufs[g], acc.at[idx.at[0, 1]], ssems[g]).wait()

    for g in range(NG):
        _gst(g, g)

    def _it(t, carry):
        r = t * NG
        for g in range(NG):
            _gwt(g)
            _sst(g, r + g)
        for g in range(NG):
            _swt(g)
            _gst(g, r + NG + g)
        return carry
    lax.fori_loop(0, nrows // NG - 1, _it, 0)
    r = nrows - NG
    for g in range(NG):
        _gwt(g)
        _sst(g, r + g)
    for g in range(NG):
        _swt(g)


def _copy_out(acc, out, c, s):
    wb = 624
    _sync_copy(acc.at[pl.ds(s * wb, wb)],
               out.at[pl.ds(c * N + s * wb, wb)])

    @pl.when(s == NTILES - 1)
    def _():
        _sync_copy(acc.at[pl.ds(NTILES * wb, N - NTILES * wb)],
                   out.at[pl.ds(c * N + NTILES * wb, N - NTILES * wb)])


def _zero_acc(acc, zb, w, s):
    z16 = jnp.zeros((16,), jnp.float32)

    def _zr(i, carry):
        for k in range(w // 16):
            zb[i, pl.ds(k * 16, 16)] = z16
        return carry
    lax.fori_loop(0, EPB, _zr, 0)
    z0 = s * (ACC_ROWS // NTILES)
    for i in range((ACC_ROWS // NTILES) // EPB):
        _sync_copy(zb, acc.at[pl.ds(z0 + i * EPB, EPB)])


def _msg_body(w, h, sdx, out, acc, idx,
              b0, b1, b2, b3, b4, b5, b6, b7, gsem, ssem):
    c = lax.axis_index("c")
    s = lax.axis_index("s")
    bufs = [b0, b1, b2, b3, b4, b5, b6, b7]
    gsems = [gsem.at[g] for g in range(NG)]
    ssems = [ssem.at[g] for g in range(NG)]
    _zero_acc(acc, b0, w, s)
    plsc.subcore_barrier()

    def _ph(p, carry):
        r0 = s * BLK_PER_TILE + p * PH_W
        _sync_copy(sdx.at[c, pl.ds(r0, PH_W)], idx)
        _msg_phase(h, acc, idx, bufs, gsems, ssems, PH_W)
        return carry
    lax.fori_loop(0, BLK_PER_TILE // PH_W, _ph, 0)
    plsc.subcore_barrier()
    _copy_out(acc, out, c, s)


@functools.partial(jax.jit, static_argnums=0)
def _msg_kernel(w, h, sdx):
    body = functools.partial(_msg_body, w)
    return pl.kernel(
        body,
        out_type=jax.ShapeDtypeStruct((2 * N, w), jnp.float32),
        mesh=_sc_mesh(),
        scratch_types=[
            pltpu.VMEM_SHARED((ACC_ROWS, w), jnp.float32),
            pltpu.VMEM((PH_W, 2, EPB), jnp.int32),
        ] + [pltpu.VMEM((EPB, w), jnp.float32)] * NG + [
            pltpu.SemaphoreType.DMA((NG,)),
            pltpu.SemaphoreType.DMA((NG,)),
        ],
    )(h, sdx)


def _msg_es_body(h, sdx, out, acc, idx,
                 b0, b1, b2, b3, b4, b5, b6, b7, gsem, ssem):
    c = lax.axis_index("c")
    s = lax.axis_index("s")
    bufs = [b0, b1, b2, b3, b4, b5, b6, b7]
    gsems = [gsem.at[g] for g in range(NG)]
    ssems = [ssem.at[g] for g in range(NG)]
    _zero_acc(acc, b0, 128, s)
    plsc.subcore_barrier()
    base = c * (BLK // 2) + s * BLK_ES

    def _ph(p, carry):
        r0 = base + p * PH_ES
        _sync_copy(sdx.at[0, pl.ds(r0, PH_ES)], idx)
        _msg_phase(h, acc, idx, bufs, gsems, ssems, PH_ES)
        return carry
    lax.fori_loop(0, BLK_ES // PH_ES, _ph, 0)
    plsc.subcore_barrier()
    _copy_out(acc, out, c, s)


@jax.jit
def _msg_es_kernel(h, sdx):
    return pl.kernel(
        _msg_es_body,
        out_type=jax.ShapeDtypeStruct((2 * N, 128), jnp.float32),
        mesh=_sc_mesh(),
        scratch_types=[
            pltpu.VMEM_SHARED((ACC_ROWS, 128), jnp.float32),
            pltpu.VMEM((PH_ES, 2, EPB), jnp.int32),
        ] + [pltpu.VMEM((EPB, 128), jnp.float32)] * NG + [
            pltpu.SemaphoreType.DMA((NG,)),
            pltpu.SemaphoreType.DMA((NG,)),
        ],
    )(h, sdx)


def _rsqrt0(d):
    return jnp.where(d > 0, lax.rsqrt(d), 0.0)


BN = 2000
NB = N // BN


def _lin1_body(x_ref, w_ref, o_ref):
    o_ref[...] = jnp.dot(x_ref[...], w_ref[...],
                         preferred_element_type=jnp.float32)


@jax.jit
def _lin1_kernel(x, W1):
    f = W1.shape[0]
    return pl.pallas_call(
        _lin1_body,
        grid=(2, NB),
        in_specs=[
            pl.BlockSpec((BN, f), lambda c, n: (n, 0)),
            pl.BlockSpec((f, 128), lambda c, n: (0, c)),
        ],
        out_specs=pl.BlockSpec((BN, 128), lambda c, n: (c * NB + n, 0)),
        out_shape=jax.ShapeDtypeStruct((2 * N, 128), jnp.float32),
    )(x, W1)


def _mid_body(wh, a0_ref, a1_ref, ni_ref, no_ref, b_ref, w_ref, o_ref):
    ni = _rsqrt0(ni_ref[...])
    no = _rsqrt0(no_ref[...])
    b = b_ref[...]
    t0 = jnp.maximum(a0_ref[...] * ni + b[:, :128], 0.0) * no
    t1 = jnp.maximum(a1_ref[...] * ni + b[:, 128:], 0.0) * no
    t = jnp.concatenate([t0, t1], axis=1)
    o_ref[...] = jnp.dot(t, w_ref[...], preferred_element_type=jnp.float32)


@functools.partial(jax.jit, static_argnums=0)
def _mid_kernel(wh, agg, nin, nout, b, W):
    body = functools.partial(_mid_body, wh)
    if wh == 128:
        return pl.pallas_call(
            body,
            grid=(2, NB),
            in_specs=[
                pl.BlockSpec((BN, 128), lambda c, n: (n, 0)),
                pl.BlockSpec((BN, 128), lambda c, n: (NB + n, 0)),
                pl.BlockSpec((BN, 1), lambda c, n: (n, 0)),
                pl.BlockSpec((BN, 1), lambda c, n: (n, 0)),
                pl.BlockSpec((1, 256), lambda c, n: (0, 0)),
                pl.BlockSpec((256, wh), lambda c, n: (0, c)),
            ],
            out_specs=pl.BlockSpec((BN, wh), lambda c, n: (c * NB + n, 0)),
            out_shape=jax.ShapeDtypeStruct((2 * N, wh), jnp.float32),
        )(agg, agg, nin, nout, b, W)
    return pl.pallas_call(
        body,
        grid=(NB,),
        in_specs=[
            pl.BlockSpec((BN, 128), lambda n: (n, 0)),
            pl.BlockSpec((BN, 128), lambda n: (NB + n, 0)),
            pl.BlockSpec((BN, 1), lambda n: (n, 0)),
            pl.BlockSpec((BN, 1), lambda n: (n, 0)),
            pl.BlockSpec((1, 256), lambda n: (0, 0)),
            pl.BlockSpec((256, 2 * wh), lambda n: (0, 0)),
        ],
        out_specs=pl.BlockSpec((BN, 2 * wh), lambda n: (n, 0)),
        out_shape=jax.ShapeDtypeStruct((N, 2 * wh), jnp.float32),
    )(agg, agg, nin, nout, b, W)


def _fin_body(a0_ref, a1_ref, ni_ref, b_ref, o_ref):
    ni = _rsqrt0(ni_ref[...])
    b = b_ref[...]
    t = (a0_ref[...] + a1_ref[...])[:, :64]
    o_ref[...] = t * ni + b


@jax.jit
def _fin_kernel(agg, nin, b):
    return pl.pallas_call(
        _fin_body,
        grid=(NB,),
        in_specs=[
            pl.BlockSpec((BN, 128), lambda n: (n, 0)),
            pl.BlockSpec((BN, 128), lambda n: (NB + n, 0)),
            pl.BlockSpec((BN, 1), lambda n: (n, 0)),
            pl.BlockSpec((1, 64), lambda n: (0, 0)),
        ],
        out_specs=pl.BlockSpec((BN, 64), lambda n: (n, 0)),
        out_shape=jax.ShapeDtypeStruct((N, 64), jnp.float32),
    )(agg, agg, nin, b)


NPADE = ROWS_PAD * LANES - E


def _prep_graph(edge_index):
    src = edge_index[0].astype(jnp.int32)
    dst = edge_index[1].astype(jnp.int32)
    i = jnp.arange(NPADE, dtype=jnp.int32)
    src_p = jnp.concatenate([src, (i * 97) % N])
    dst_p = jnp.concatenate([dst, N + (i % (ACC_ROWS - N))])
    sdx = jnp.stack([
        jnp.stack([src_p.reshape(BLK, EPB), dst_p.reshape(BLK, EPB)], 1),
        jnp.stack([(src_p + N).reshape(BLK, EPB), dst_p.reshape(BLK, EPB)],
                  1)])
    src_d = jnp.concatenate([src, N + (i % 240)]).reshape(ROWS_PAD, LANES)
    dst_d = jnp.concatenate([dst, N + (i % 240)]).reshape(ROWS_PAD, LANES)
    return sdx, jnp.stack([src_d, dst_d])


def kernel(features, edge_index1, edge_index2, W1, b1, W2, b2, W3, b3):
    sdx1, deg_ei1 = _prep_graph(edge_index1)
    sdx2, deg_ei2 = _prep_graph(edge_index2)
    degs = _deg_kernel(jnp.stack([deg_ei1, deg_ei2])).reshape(2, 2, DEG_PAD)
    dout1 = degs[0, 0, :N, None]
    din1 = degs[0, 1, :N, None]
    dout2 = degs[1, 0, :N, None]
    din2 = degs[1, 1, :N, None]

    mm1 = _lin1_kernel(features, W1)
    h1 = (mm1.reshape(2, N, 128) * _rsqrt0(dout2)[None]
          ).reshape(2 * N, 128)
    agg1 = _msg_kernel(128, h1, sdx2)
    h2 = _mid_kernel(128, agg1, din2, dout1, b1[None, :], W2)
    agg2 = _msg_kernel(128, h2, sdx1)
    h3 = _mid_kernel(32, agg2, din1, dout1, b2[None, :], W3)
    h3p = jnp.pad(h3, ((0, 0), (0, 64)))
    agg3 = _msg_es_kernel(h3p, sdx1)
    return _fin_kernel(agg3, din1, b3[None, :])


# speedup vs baseline: 7.7850x; 1.0345x over previous
import functools

import jax
import jax.numpy as jnp
from jax import lax
from jax.experimental import pallas as pl
from jax.experimental.pallas import tpu as pltpu
from jax.experimental.pallas import tpu_sc as plsc

N = 10000
E = 320000
LANES = 128
ROWS = E // LANES
ROWS_PAD = 2560
NTILES = 16
ROWS_PER_TILE = ROWS_PAD // NTILES
SB = 8
NSB = ROWS_PER_TILE // SB
ACC_ROWS = 10240
DEG_PAD = 10240


def _sync_copy(src, dst, *, add=False):
    def _inner(sem):
        desc = pltpu.make_async_copy(src, dst, sem)
        desc.start(add=add)
        desc.wait()
    pl.run_scoped(_inner, sem=pltpu.SemaphoreType.DMA(()))


def _sc_mesh():
    return plsc.VectorSubcoreMesh(core_axis_name="c", subcore_axis_name="s")


DEG_CH = 8


def _deg_body(s1, d1, s2, d2, out, degs, degd, sidx, didx, ones, zbuf, sem):
    c = lax.axis_index("c")
    s = lax.axis_index("s")
    o16 = jnp.ones((16,), jnp.float32)
    z16 = jnp.zeros((16,), jnp.float32)
    for i in range(LANES // 16):
        ones[pl.ds(i * 16, 16)] = o16

    def _zb(i, carry):
        zbuf[pl.ds(i * 16, 16)] = z16
        return carry
    lax.fori_loop(0, 640 // 16, _zb, 0)
    _sync_copy(zbuf, degs.at[pl.ds(s * 640, 640)])
    _sync_copy(zbuf, degd.at[pl.ds(s * 640, 640)])
    r0 = s * ROWS_PER_TILE

    @pl.when(c == 0)
    def _():
        _sync_copy(s1.at[pl.ds(r0, ROWS_PER_TILE)], sidx)
        _sync_copy(d1.at[pl.ds(r0, ROWS_PER_TILE)], didx)

    @pl.when(c == 1)
    def _():
        _sync_copy(s2.at[pl.ds(r0, ROWS_PER_TILE)], sidx)
        _sync_copy(d2.at[pl.ds(r0, ROWS_PER_TILE)], didx)
    plsc.subcore_barrier()

    def _ch(i, carry):
        j0 = i * DEG_CH
        for j in range(DEG_CH):
            pltpu.make_async_copy(ones, degs.at[sidx.at[j0 + j]], sem
                                  ).start(add=True)
            pltpu.make_async_copy(ones, degd.at[didx.at[j0 + j]], sem
                                  ).start(add=True)
        for j in range(DEG_CH):
            pltpu.make_async_copy(ones, degs.at[sidx.at[j0 + j]], sem).wait()
            pltpu.make_async_copy(ones, degd.at[didx.at[j0 + j]], sem).wait()
        return carry
    lax.fori_loop(0, ROWS_PER_TILE // DEG_CH, _ch, 0)
    plsc.subcore_barrier()
    _sync_copy(degs.at[pl.ds(s * 640, 640)],
               out.at[pl.ds((c * 2 + 0) * DEG_PAD + s * 640, 640)])
    _sync_copy(degd.at[pl.ds(s * 640, 640)],
               out.at[pl.ds((c * 2 + 1) * DEG_PAD + s * 640, 640)])


@jax.jit
def _deg_kernel(s1, d1, s2, d2):
    return pl.kernel(
        _deg_body,
        out_type=jax.ShapeDtypeStruct((4 * DEG_PAD,), jnp.float32),
        mesh=_sc_mesh(),
        scratch_types=[
            pltpu.VMEM_SHARED((DEG_PAD,), jnp.float32),
            pltpu.VMEM_SHARED((DEG_PAD,), jnp.float32),
            pltpu.VMEM((ROWS_PER_TILE, LANES), jnp.int32),
            pltpu.VMEM((ROWS_PER_TILE, LANES), jnp.int32),
            pltpu.VMEM((LANES,), jnp.float32),
            pltpu.VMEM((640,), jnp.float32),
            pltpu.SemaphoreType.DMA(()),
        ],
    )(s1, d1, s2, d2)


EPB = 32
NG = 8
BLK = ROWS_PAD * LANES // EPB
BLK_PER_TILE = BLK // NTILES
PH_W = 2048 // EPB
BLK_ES = BLK // 2 // NTILES
PH_ES = PH_W


def _msg_phase(h, acc, sidx, didx, bufs, gsems, ssems, nrows):
    def _gst(g, r):
        pltpu.make_async_copy(h.at[sidx.at[r]], bufs[g], gsems[g]).start()

    def _gwt(g):
        pltpu.make_async_copy(h.at[sidx.at[0]], bufs[g], gsems[g]).wait()

    def _sst(g, r):
        pltpu.make_async_copy(bufs[g], acc.at[didx.at[r]], ssems[g]
                              ).start(add=True)

    def _swt(g):
        pltpu.make_async_copy(bufs[g], acc.at[didx.at[0]], ssems[g]).wait()

    for g in range(NG):
        _gst(g, g)

    def _it(t, carry):
        r = t * NG
        for g in range(NG):
            _gwt(g)
            _sst(g, r + g)
        for g in range(NG):
            _swt(g)
            _gst(g, r + NG + g)
        return carry
    lax.fori_loop(0, nrows // NG - 1, _it, 0)
    r = nrows - NG
    for g in range(NG):
        _gwt(g)
        _sst(g, r + g)
    for g in range(NG):
        _swt(g)


def _copy_out(acc, out, c, s):
    wb = 624
    _sync_copy(acc.at[pl.ds(s * wb, wb)],
               out.at[pl.ds(c * N + s * wb, wb)])

    @pl.when(s == NTILES - 1)
    def _():
        _sync_copy(acc.at[pl.ds(NTILES * wb, N - NTILES * wb)],
                   out.at[pl.ds(c * N + NTILES * wb, N - NTILES * wb)])


def _zero_acc(acc, zb, w, s):
    z16 = jnp.zeros((16,), jnp.float32)

    def _zr(i, carry):
        for k in range(w // 16):
            zb[i, pl.ds(k * 16, 16)] = z16
        return carry
    lax.fori_loop(0, EPB, _zr, 0)
    z0 = s * (ACC_ROWS // NTILES)
    for i in range((ACC_ROWS // NTILES) // EPB):
        _sync_copy(zb, acc.at[pl.ds(z0 + i * EPB, EPB)])


def _msg_body(w, h, srcp, dstp, out, acc, sidx, didx, *rest):
    c = lax.axis_index("c")
    s = lax.axis_index("s")
    bufs = list(rest[:NG])
    gsem, ssem = rest[NG], rest[NG + 1]
    gsems = [gsem.at[g] for g in range(NG)]
    ssems = [ssem.at[g] for g in range(NG)]
    _zero_acc(acc, bufs[0], w, s)
    plsc.subcore_barrier()
    cn = jnp.zeros((16,), jnp.int32) + c * N

    def _ph(p, carry):
        r0 = s * BLK_PER_TILE + p * PH_W
        _sync_copy(srcp.at[pl.ds(r0, PH_W)], sidx)
        _sync_copy(dstp.at[pl.ds(r0, PH_W)], didx)

        @pl.when(c == 1)
        def _():
            def _add(i, carry2):
                for k in range(EPB // 16):
                    sidx[i, pl.ds(k * 16, 16)] = (
                        sidx[i, pl.ds(k * 16, 16)] + cn)
                return carry2
            lax.fori_loop(0, PH_W, _add, 0)
        _msg_phase(h, acc, sidx, didx, bufs, gsems, ssems, PH_W)
        return carry
    lax.fori_loop(0, BLK_PER_TILE // PH_W, _ph, 0)
    plsc.subcore_barrier()
    _copy_out(acc, out, c, s)


@functools.partial(jax.jit, static_argnums=0)
def _msg_kernel(w, h, srcp, dstp):
    body = functools.partial(_msg_body, w)
    return pl.kernel(
        body,
        out_type=jax.ShapeDtypeStruct((2 * N, w), jnp.float32),
        mesh=_sc_mesh(),
        scratch_types=[
            pltpu.VMEM_SHARED((ACC_ROWS, w), jnp.float32),
            pltpu.VMEM((PH_W, EPB), jnp.int32),
            pltpu.VMEM((PH_W, EPB), jnp.int32),
        ] + [pltpu.VMEM((EPB, w), jnp.float32)] * NG + [
            pltpu.SemaphoreType.DMA((NG,)),
            pltpu.SemaphoreType.DMA((NG,)),
        ],
    )(h, srcp, dstp)


def _msg_es_body(h, srcp, dstp, out, acc, sidx, didx, *rest):
    c = lax.axis_index("c")
    s = lax.axis_index("s")
    bufs = list(rest[:NG])
    gsem, ssem = rest[NG], rest[NG + 1]
    gsems = [gsem.at[g] for g in range(NG)]
    ssems = [ssem.at[g] for g in range(NG)]
    _zero_acc(acc, bufs[0], 128, s)
    plsc.subcore_barrier()
    base = c * (BLK // 2) + s * BLK_ES

    def _ph(p, carry):
        r0 = base + p * PH_ES
        _sync_copy(srcp.at[pl.ds(r0, PH_ES)], sidx)
        _sync_copy(dstp.at[pl.ds(r0, PH_ES)], didx)
        _msg_phase(h, acc, sidx, didx, bufs, gsems, ssems, PH_ES)
        return carry
    lax.fori_loop(0, BLK_ES // PH_ES, _ph, 0)
    plsc.subcore_barrier()
    _copy_out(acc, out, c, s)


@jax.jit
def _msg_es_kernel(h, srcp, dstp):
    return pl.kernel(
        _msg_es_body,
        out_type=jax.ShapeDtypeStruct((2 * N, 128), jnp.float32),
        mesh=_sc_mesh(),
        scratch_types=[
            pltpu.VMEM_SHARED((ACC_ROWS, 128), jnp.float32),
            pltpu.VMEM((PH_ES, EPB), jnp.int32),
            pltpu.VMEM((PH_ES, EPB), jnp.int32),
        ] + [pltpu.VMEM((EPB, 128), jnp.float32)] * NG + [
            pltpu.SemaphoreType.DMA((NG,)),
            pltpu.SemaphoreType.DMA((NG,)),
        ],
    )(h, srcp, dstp)


def _rsqrt0(d):
    return jnp.where(d > 0, lax.rsqrt(d), 0.0)


BN = 2000
NB = N // BN


def _lin1_body(x_ref, w_ref, o_ref):
    o_ref[...] = jnp.dot(x_ref[...], w_ref[...],
                         preferred_element_type=jnp.float32)


@jax.jit
def _lin1_kernel(x, W1):
    f = W1.shape[0]
    return pl.pallas_call(
        _lin1_body,
        grid=(2, NB),
        in_specs=[
            pl.BlockSpec((BN, f), lambda c, n: (n, 0)),
            pl.BlockSpec((f, 128), lambda c, n: (0, c)),
        ],
        out_specs=pl.BlockSpec((BN, 128), lambda c, n: (c * NB + n, 0)),
        out_shape=jax.ShapeDtypeStruct((2 * N, 128), jnp.float32),
    )(x, W1)


def _mid_body(wh, a0_ref, a1_ref, ni_ref, no_ref, b_ref, w_ref, o_ref):
    ni = _rsqrt0(ni_ref[...])
    no = _rsqrt0(no_ref[...])
    b = b_ref[...]
    t0 = jnp.maximum(a0_ref[...] * ni + b[:, :128], 0.0) * no
    t1 = jnp.maximum(a1_ref[...] * ni + b[:, 128:], 0.0) * no
    t = jnp.concatenate([t0, t1], axis=1)
    o_ref[...] = jnp.dot(t, w_ref[...], preferred_element_type=jnp.float32)


@functools.partial(jax.jit, static_argnums=0)
def _mid_kernel(wh, agg, nin, nout, b, W):
    body = functools.partial(_mid_body, wh)
    if wh == 128:
        return pl.pallas_call(
            body,
            grid=(2, NB),
            in_specs=[
                pl.BlockSpec((BN, 128), lambda c, n: (n, 0)),
                pl.BlockSpec((BN, 128), lambda c, n: (NB + n, 0)),
                pl.BlockSpec((BN, 1), lambda c, n: (n, 0)),
                pl.BlockSpec((BN, 1), lambda c, n: (n, 0)),
                pl.BlockSpec((1, 256), lambda c, n: (0, 0)),
                pl.BlockSpec((256, wh), lambda c, n: (0, c)),
            ],
            out_specs=pl.BlockSpec((BN, wh), lambda c, n: (c * NB + n, 0)),
            out_shape=jax.ShapeDtypeStruct((2 * N, wh), jnp.float32),
        )(agg, agg, nin, nout, b, W)
    return pl.pallas_call(
        body,
        grid=(NB,),
        in_specs=[
            pl.BlockSpec((BN, 128), lambda n: (n, 0)),
            pl.BlockSpec((BN, 128), lambda n: (NB + n, 0)),
            pl.BlockSpec((BN, 1), lambda n: (n, 0)),
            pl.BlockSpec((BN, 1), lambda n: (n, 0)),
            pl.BlockSpec((1, 256), lambda n: (0, 0)),
            pl.BlockSpec((256, 2 * wh), lambda n: (0, 0)),
        ],
        out_specs=pl.BlockSpec((BN, 2 * wh), lambda n: (n, 0)),
        out_shape=jax.ShapeDtypeStruct((N, 2 * wh), jnp.float32),
    )(agg, agg, nin, nout, b, W)


def _fin_body(a0_ref, a1_ref, ni_ref, b_ref, o_ref):
    ni = _rsqrt0(ni_ref[...])
    b = b_ref[...]
    t = (a0_ref[...] + a1_ref[...])[:, :64]
    o_ref[...] = t * ni + b


@jax.jit
def _fin_kernel(agg, nin, b):
    return pl.pallas_call(
        _fin_body,
        grid=(NB,),
        in_specs=[
            pl.BlockSpec((BN, 128), lambda n: (n, 0)),
            pl.BlockSpec((BN, 128), lambda n: (NB + n, 0)),
            pl.BlockSpec((BN, 1), lambda n: (n, 0)),
            pl.BlockSpec((1, 64), lambda n: (0, 0)),
        ],
        out_specs=pl.BlockSpec((BN, 64), lambda n: (n, 0)),
        out_shape=jax.ShapeDtypeStruct((N, 64), jnp.float32),
    )(agg, agg, nin, b)


NPADE = ROWS_PAD * LANES - E


def _prep_graph(edge_index):
    src = edge_index[0].astype(jnp.int32)
    dst = edge_index[1].astype(jnp.int32)
    i = jnp.arange(NPADE, dtype=jnp.int32)
    srcp = jnp.concatenate([src, (i * 97) % N]).reshape(BLK, EPB)
    dstp = jnp.concatenate([dst, N + (i % (ACC_ROWS - N))]).reshape(BLK, EPB)
    sdeg = jnp.concatenate([src, N + (i % 240)]).reshape(ROWS_PAD, LANES)
    return srcp, dstp, sdeg


def kernel(features, edge_index1, edge_index2, W1, b1, W2, b2, W3, b3):
    srcp1, dstp1, sdeg1 = _prep_graph(edge_index1)
    srcp2, dstp2, sdeg2 = _prep_graph(edge_index2)
    degs = _deg_kernel(sdeg1, dstp1.reshape(ROWS_PAD, LANES),
                       sdeg2, dstp2.reshape(ROWS_PAD, LANES)
                       ).reshape(2, 2, DEG_PAD)
    dout1 = degs[0, 0, :N, None]
    din1 = degs[0, 1, :N, None]
    dout2 = degs[1, 0, :N, None]
    din2 = degs[1, 1, :N, None]

    mm1 = _lin1_kernel(features, W1)
    h1 = (mm1.reshape(2, N, 128) * _rsqrt0(dout2)[None]
          ).reshape(2 * N, 128)
    agg1 = _msg_kernel(128, h1, srcp2, dstp2)
    h2 = _mid_kernel(128, agg1, din2, dout1, b1[None, :], W2)
    agg2 = _msg_kernel(128, h2, srcp1, dstp1)
    h3 = _mid_kernel(32, agg2, din1, dout1, b2[None, :], W3)
    h3p = jnp.pad(h3, ((0, 0), (0, 64)))
    agg3 = _msg_es_kernel(h3p, srcp1, dstp1)
    return _fin_kernel(agg3, din1, b3[None, :])
